# baseline (speedup 1.0000x reference)
import functools

import jax
import jax.numpy as jnp
from jax import lax
from jax.experimental import pallas as pl
from jax.experimental.pallas import tpu as pltpu
from jax.experimental.pallas import tpu_sc as plsc

H = 128
N = 10000
E = 320000
N2 = 2000
NS = 400
G = 16
C = 10

NB = 1000
NBLK = N // NB
EROW = 128
NW = 32
GPW = 80
ERP = NW * GPW
EPAD = ERP * EROW - E
NP = N + 8


def _mm(a, b, dims):
    return lax.dot_general(a, b, dimension_numbers=(dims, ((), ())),
                           preferred_element_type=jnp.float32,
                           precision=lax.Precision.HIGHEST)


def _sig(x):
    return 1.0 / (1.0 + jnp.exp(-x))


def _tanh(x):
    return 2.0 / (1.0 + jnp.exp(-2.0 * x)) - 1.0


def _mlp2(h, w1, b1, w2, b2):
    hid = jnp.maximum(_mm(h, w1, ((1,), (0,))) + b1, 0.0)
    return _mm(hid, w2, ((1,), (0,))) + b2


def _seg_onehot(ids_row, nseg):
    L = ids_row.shape[1]
    return (lax.broadcasted_iota(jnp.int32, (nseg, L), 0) == ids_row
            ).astype(jnp.float32)


def _prelude_body(z_ref, seg_ref, emb_ref, x_ref, pxW_ref, pxb_ref,
                  iew1_ref, ieb1_ref, iew2_ref, ieb2_ref,
                  inw1_ref, inb1_ref, inw2_ref, inb2_ref,
                  c0w_ref, s2s_ref, s2g_ref,
                  zf_ref, m0_ref, xf_ref, ge0_ref, acc_ref):
    i = pl.program_id(0)

    @pl.when(i == 0)
    def _():
        acc_ref[...] = jnp.zeros_like(acc_ref)

    oh = _seg_onehot(z_ref[0], 100)
    zf = jnp.maximum(_mm(oh, emb_ref[...], ((0,), (0,))), 0.0)
    zf_ref[...] = zf
    m0_ref[...] = _mm(zf, c0w_ref[...], ((1,), (0,)))
    soh = _seg_onehot(seg_ref[0], N2)
    acc_ref[...] += _mm(soh, zf, ((1,), (0,)))

    @pl.when(i == NBLK - 1)
    def _():
        xf = jnp.maximum(_mm(x_ref[...], pxW_ref[...], ((1,), (0,)))
                         + pxb_ref[...], 0.0)
        xf_ref[...] = xf
        ne = _mlp2(acc_ref[...], iew1_ref[...], ieb1_ref[...],
                   iew2_ref[...], ieb2_ref[...])
        se = _mm(_seg_onehot(s2s_ref[...], NS), ne, ((1,), (0,)))
        se = _mlp2(se, inw1_ref[...], inb1_ref[...],
                   inw2_ref[...], inb2_ref[...])
        ge0_ref[...] = _mm(_seg_onehot(s2g_ref[...], G), se * xf,
                           ((1,), (0,)))


def _prelude(z3, n2s3, emb, x, pxW, pxb, iew1, ieb1, iew2, ieb2,
             inw1, inb1, inw2, inb2, c0w, s2s, s2g):
    full = lambda shp: pl.BlockSpec(shp, lambda i: (0,) * len(shp))
    return pl.pallas_call(
        _prelude_body,
        grid=(NBLK,),
        in_specs=[
            pl.BlockSpec((1, 1, NB), lambda i: (i, 0, 0)),
            pl.BlockSpec((1, 1, NB), lambda i: (i, 0, 0)),
            full((100, H)), full((NS, H)), full((H, H)), full((1, H)),
            full((H, 2 * H)), full((1, 2 * H)), full((2 * H, H)), full((1, H)),
            full((H, 2 * H)), full((1, 2 * H)), full((2 * H, H)), full((1, H)),
            full((H, H)), full((1, N2)), full((1, NS)),
        ],
        out_specs=[
            pl.BlockSpec((NB, H), lambda i: (i, 0)),
            pl.BlockSpec((NB, H), lambda i: (i, 0)),
            full((NS, H)), full((G, H)),
        ],
        out_shape=[
            jax.ShapeDtypeStruct((N, H), jnp.float32),
            jax.ShapeDtypeStruct((N, H), jnp.float32),
            jax.ShapeDtypeStruct((NS, H), jnp.float32),
            jax.ShapeDtypeStruct((G, H), jnp.float32),
        ],
        scratch_shapes=[pltpu.VMEM((N2, H), jnp.float32)],
    )(z3, n2s3, emb, x, pxW, pxb, iew1, ieb1, iew2, ieb2,
      inw1, inb1, inw2, inb2, c0w, s2s, s2g)


def _edge_agg(m, src3, dst3, zeros_n):
    mesh = plsc.VectorSubcoreMesh(core_axis_name="c", subcore_axis_name="s")

    @functools.partial(
        pl.kernel,
        mesh=mesh,
        out_type=jax.ShapeDtypeStruct((2, N, H), jnp.float32),
        scratch_types=[
            pltpu.VMEM((EROW,), jnp.int32),
            pltpu.VMEM((EROW,), jnp.int32),
            pltpu.VMEM((EROW,), jnp.int32),
            pltpu.VMEM((EROW,), jnp.int32),
            pltpu.VMEM((EROW, H), jnp.float32),
            pltpu.VMEM((EROW, H), jnp.float32),
            pltpu.VMEM_SHARED((NP, H), jnp.float32),
            pltpu.SemaphoreType.DMA,
            pltpu.SemaphoreType.DMA,
        ],
    )
    def k(m_hbm, src_hbm, dst_hbm, z_hbm, out_hbm, src_a, dst_a, src_b,
          dst_b, rows_a, rows_b, acc, sem_a, sem_b):
        c = lax.axis_index("c")
        s = lax.axis_index("s")
        w = s * 2 + c
        rpt = 624
        ztail = NP - 16 * rpt

        pltpu.sync_copy(z_hbm.at[pl.ds(s * rpt, rpt)],
                        acc.at[pl.ds(s * rpt, rpt)])

        @pl.when(s == 15)
        def _():
            pltpu.sync_copy(z_hbm.at[pl.ds(16 * rpt, ztail)],
                            acc.at[pl.ds(16 * rpt, ztail)])

        plsc.subcore_barrier()

        def load_idx(g, sv, dv):
            pltpu.sync_copy(src_hbm.at[w, g], sv)
            pltpu.sync_copy(dst_hbm.at[w, g], dv)

        load_idx(0, src_a, dst_a)
        pltpu.async_copy(m_hbm.at[src_a], rows_a, sem_a)

        def body(t, carry):
            ga = 2 * t
            load_idx(ga + 1, src_b, dst_b)
            pltpu.async_copy(m_hbm.at[src_b], rows_b, sem_b)
            pltpu.make_async_copy(m_hbm.at[src_a], rows_a, sem_a).wait()
            pltpu.sync_copy(rows_a, acc.at[dst_a], add=True)
            load_idx(jnp.minimum(ga + 2, GPW - 1), src_a, dst_a)
            pltpu.async_copy(m_hbm.at[src_a], rows_a, sem_a)
            pltpu.make_async_copy(m_hbm.at[src_b], rows_b, sem_b).wait()
            pltpu.sync_copy(rows_b, acc.at[dst_b], add=True)
            return carry

        lax.fori_loop(0, GPW // 2, body, 0)
        pltpu.make_async_copy(m_hbm.at[src_a], rows_a, sem_a).wait()

        plsc.subcore_barrier()
        wtail = N - 16 * rpt
        pltpu.sync_copy(acc.at[pl.ds(s * rpt, rpt)],
                        out_hbm.at[c, pl.ds(s * rpt, rpt)])

        @pl.when(s == 15)
        def _():
            pltpu.sync_copy(acc.at[pl.ds(16 * rpt, wtail)],
                            out_hbm.at[c, pl.ds(16 * rpt, wtail)])

    return k(m, src3, dst3, zeros_n)


def _conv_body(has_next, final, *refs):
    if final:
        (h_ref, p_ref, seg_ref, wih_ref, whh_ref, bih_ref, bhh_ref,
         ew1_ref, eb1_ref, ew2_ref, eb2_ref,
         nw1_ref, nb1_ref, nw2_ref, nb2_ref,
         xf_ref, s2s_ref, s2g_ref, ge0_ref, ge1_ref,
         pw1_ref, pb1_ref, pw2_ref, pb2_ref,
         pred_ref, acc_ref) = refs
    else:
        (h_ref, p_ref, seg_ref, wih_ref, whh_ref, bih_ref, bhh_ref,
         ew1_ref, eb1_ref, ew2_ref, eb2_ref,
         nw1_ref, nb1_ref, nw2_ref, nb2_ref,
         xf_ref, s2s_ref, s2g_ref, cwn_ref,
         hn_ref, mn_ref, ge_ref, acc_ref) = refs

    i = pl.program_id(0)

    @pl.when(i == 0)
    def _():
        acc_ref[...] = jnp.zeros_like(acc_ref)

    h = h_ref[...]
    agg = p_ref[0] + p_ref[1]
    gi = _mm(agg, wih_ref[...], ((1,), (1,))) + bih_ref[...]
    gh = _mm(h, whh_ref[...], ((1,), (1,))) + bhh_ref[...]
    r = _sig(gi[:, :H] + gh[:, :H])
    u = _sig(gi[:, H:2 * H] + gh[:, H:2 * H])
    nn_ = _tanh(gi[:, 2 * H:] + r * gh[:, 2 * H:])
    hn = jnp.maximum((1.0 - u) * nn_ + u * h, 0.0)

    if not final:
        hn_ref[...] = hn
    if has_next:
        mn_ref[...] = _mm(hn, cwn_ref[...], ((1,), (0,)))

    soh = _seg_onehot(seg_ref[0], N2)
    acc_ref[...] += _mm(soh, hn, ((1,), (0,)))

    @pl.when(i == NBLK - 1)
    def _():
        ne = _mlp2(acc_ref[...], ew1_ref[...], eb1_ref[...],
                   ew2_ref[...], eb2_ref[...])
        se = _mm(_seg_onehot(s2s_ref[...], NS), ne, ((1,), (0,)))
        se = _mlp2(se, nw1_ref[...], nb1_ref[...],
                   nw2_ref[...], nb2_ref[...])
        ge = _mm(_seg_onehot(s2g_ref[...], G), se * xf_ref[...],
                 ((1,), (0,)))
        if final:
            embed = ge0_ref[...] + ge1_ref[...] + ge
            hid = jnp.maximum(_mm(embed, pw1_ref[...], ((1,), (0,)))
                              + pb1_ref[...], 0.0)
            pred = _mm(hid, pw2_ref[...], ((1,), (0,))) + pb2_ref[...]
            mx = jnp.max(pred, axis=-1, keepdims=True)
            sh = pred - mx
            lse = jnp.log(jnp.sum(jnp.exp(sh), axis=-1, keepdims=True))
            pred_ref[...] = sh - lse
        else:
            ge_ref[...] = ge


def _conv_layer(h, parts, n2s3, wih, whh, bih, bhh,
                ew1, eb1, ew2, eb2, nw1, nb1, nw2, nb2,
                xf, s2s, s2g, cwn=None, finals=None):
    full = lambda shp: pl.BlockSpec(shp, lambda i: (0,) * len(shp))
    final = finals is not None
    has_next = cwn is not None
    in_specs = [
        pl.BlockSpec((NB, H), lambda i: (i, 0)),
        pl.BlockSpec((2, NB, H), lambda i: (0, i, 0)),
        pl.BlockSpec((1, 1, NB), lambda i: (i, 0, 0)),
        full((3 * H, H)), full((3 * H, H)), full((1, 3 * H)), full((1, 3 * H)),
        full((H, 2 * H)), full((1, 2 * H)), full((2 * H, H)), full((1, H)),
        full((H, 2 * H)), full((1, 2 * H)), full((2 * H, H)), full((1, H)),
        full((NS, H)), full((1, N2)), full((1, NS)),
    ]
    args = [h, parts, n2s3, wih, whh, bih, bhh,
            ew1, eb1, ew2, eb2, nw1, nb1, nw2, nb2, xf, s2s, s2g]
    if final:
        ge0, ge1, pw1, pb1, pw2, pb2 = finals
        in_specs += [full((G, H)), full((G, H)),
                     full((H, H)), full((1, H)), full((H, C)), full((1, C))]
        args += [ge0, ge1, pw1, pb1, pw2, pb2]
        out_specs = [full((G, C))]
        out_shape = [jax.ShapeDtypeStruct((G, C), jnp.float32)]
    else:
        in_specs += [full((H, H))]
        args += [cwn]
        out_specs = [pl.BlockSpec((NB, H), lambda i: (i, 0)),
                     pl.BlockSpec((NB, H), lambda i: (i, 0)),
                     full((G, H))]
        out_shape = [jax.ShapeDtypeStruct((N, H), jnp.float32),
                     jax.ShapeDtypeStruct((N, H), jnp.float32),
                     jax.ShapeDtypeStruct((G, H), jnp.float32)]
    return pl.pallas_call(
        functools.partial(_conv_body, has_next, final),
        grid=(NBLK,),
        in_specs=in_specs,
        out_specs=out_specs,
        out_shape=out_shape,
        scratch_shapes=[pltpu.VMEM((N2, H), jnp.float32)],
    )(*args)


def kernel(z, x, edge_index, batch, node_to_subgraph2, subgraph2_to_subgraph,
           subgraph_to_graph, emb, pxW, pxb, ie_w1, ie_b1, ie_w2, ie_b2,
           in_w1, in_b1, in_w2, in_b2,
           conv0_w, conv0_wih, conv0_whh, conv0_bih, conv0_bhh,
           e0_w1, e0_b1, e0_w2, e0_b2, n0_w1, n0_b1, n0_w2, n0_b2,
           conv1_w, conv1_wih, conv1_whh, conv1_bih, conv1_bhh,
           e1_w1, e1_b1, e1_w2, e1_b2, n1_w1, n1_b1, n1_w2, n1_b2,
           post_w1, post_b1, post_w2, post_b2):
    i32 = jnp.int32
    z3 = z.astype(i32).reshape(NBLK, 1, NB)
    n2s3 = node_to_subgraph2.astype(i32).reshape(NBLK, 1, NB)
    s2s = subgraph2_to_subgraph.astype(i32).reshape(1, N2)
    s2g = subgraph_to_graph.astype(i32).reshape(1, NS)
    srcp = jnp.concatenate(
        [edge_index[0].astype(i32), jnp.zeros((EPAD,), i32)])
    dstp = jnp.concatenate(
        [edge_index[1].astype(i32),
         N + (jnp.arange(EPAD, dtype=i32) % (NP - N))])
    src3 = srcp.reshape(GPW, NW, EROW).transpose(1, 0, 2)
    dst3 = dstp.reshape(GPW, NW, EROW).transpose(1, 0, 2)
    zeros_n = jnp.zeros((NP, H), jnp.float32)
    row = lambda b: b.reshape(1, -1)

    zf, m0, xf, ge0 = _prelude(
        z3, n2s3, emb, x, pxW, row(pxb),
        ie_w1, row(ie_b1), ie_w2, row(ie_b2),
        in_w1, row(in_b1), in_w2, row(in_b2), conv0_w, s2s, s2g)

    parts0 = _edge_agg(m0, src3, dst3, zeros_n)
    h1, m1, ge1 = _conv_layer(
        zf, parts0, n2s3, conv0_wih, conv0_whh, row(conv0_bih),
        row(conv0_bhh), e0_w1, row(e0_b1), e0_w2, row(e0_b2),
        n0_w1, row(n0_b1), n0_w2, row(n0_b2), xf, s2s, s2g, cwn=conv1_w)

    parts1 = _edge_agg(m1, src3, dst3, zeros_n)
    (pred,) = _conv_layer(
        h1, parts1, n2s3, conv1_wih, conv1_whh, row(conv1_bih),
        row(conv1_bhh), e1_w1, row(e1_b1), e1_w2, row(e1_b2),
        n1_w1, row(n1_b1), n1_w2, row(n1_b2), xf, s2s, s2g,
        finals=(ge0, ge1, post_w1, row(post_b1), post_w2, row(post_b2)))
    return pred


# speedup vs baseline: 1.0914x; 1.0914x over previous
import functools

import jax
import jax.numpy as jnp
from jax import lax
from jax.experimental import pallas as pl
from jax.experimental.pallas import tpu as pltpu
from jax.experimental.pallas import tpu_sc as plsc

H = 128
N = 10000
E = 320000
N2 = 2000
NS = 400
G = 16
C = 10

NB = 1000
NBLK = N // NB
EROW = 128
NW = 32
GPW = 80
ERP = NW * GPW
EPAD = ERP * EROW - E
NP = N + 8


def _mm(a, b, dims):
    return lax.dot_general(a, b, dimension_numbers=(dims, ((), ())),
                           preferred_element_type=jnp.float32,
                           precision=lax.Precision.HIGHEST)


def _sig(x):
    return 1.0 / (1.0 + jnp.exp(-x))


def _tanh(x):
    return 2.0 / (1.0 + jnp.exp(-2.0 * x)) - 1.0


def _mlp2(h, w1, b1, w2, b2):
    hid = jnp.maximum(_mm(h, w1, ((1,), (0,))) + b1, 0.0)
    return _mm(hid, w2, ((1,), (0,))) + b2


def _seg_onehot(ids_row, nseg):
    L = ids_row.shape[1]
    return (lax.broadcasted_iota(jnp.int32, (nseg, L), 0) == ids_row
            ).astype(jnp.float32)


def _prelude_body(z_ref, seg_ref, emb_ref, x_ref, pxW_ref, pxb_ref,
                  iew1_ref, ieb1_ref, iew2_ref, ieb2_ref,
                  inw1_ref, inb1_ref, inw2_ref, inb2_ref,
                  c0w_ref, s2s_ref, s2g_ref,
                  zf_ref, m0_ref, xf_ref, ge0_ref, acc_ref):
    i = pl.program_id(0)

    @pl.when(i == 0)
    def _():
        acc_ref[...] = jnp.zeros_like(acc_ref)

    oh = _seg_onehot(z_ref[0], 100)
    zf = jnp.maximum(_mm(oh, emb_ref[...], ((0,), (0,))), 0.0)
    zf_ref[...] = zf
    m0_ref[...] = _mm(zf, c0w_ref[...], ((1,), (0,)))
    soh = _seg_onehot(seg_ref[0], N2)
    acc_ref[...] += _mm(soh, zf, ((1,), (0,)))

    @pl.when(i == NBLK - 1)
    def _():
        xf = jnp.maximum(_mm(x_ref[...], pxW_ref[...], ((1,), (0,)))
                         + pxb_ref[...], 0.0)
        xf_ref[...] = xf
        ne = _mlp2(acc_ref[...], iew1_ref[...], ieb1_ref[...],
                   iew2_ref[...], ieb2_ref[...])
        se = _mm(_seg_onehot(s2s_ref[...], NS), ne, ((1,), (0,)))
        se = _mlp2(se, inw1_ref[...], inb1_ref[...],
                   inw2_ref[...], inb2_ref[...])
        ge0_ref[...] = _mm(_seg_onehot(s2g_ref[...], G), se * xf,
                           ((1,), (0,)))


def _prelude(z3, n2s3, emb, x, pxW, pxb, iew1, ieb1, iew2, ieb2,
             inw1, inb1, inw2, inb2, c0w, s2s, s2g):
    full = lambda shp: pl.BlockSpec(shp, lambda i: (0,) * len(shp))
    return pl.pallas_call(
        _prelude_body,
        grid=(NBLK,),
        in_specs=[
            pl.BlockSpec((1, 1, NB), lambda i: (i, 0, 0)),
            pl.BlockSpec((1, 1, NB), lambda i: (i, 0, 0)),
            full((100, H)), full((NS, H)), full((H, H)), full((1, H)),
            full((H, 2 * H)), full((1, 2 * H)), full((2 * H, H)), full((1, H)),
            full((H, 2 * H)), full((1, 2 * H)), full((2 * H, H)), full((1, H)),
            full((H, H)), full((1, N2)), full((1, NS)),
        ],
        out_specs=[
            pl.BlockSpec((NB, H), lambda i: (i, 0)),
            pl.BlockSpec((NB, H), lambda i: (i, 0)),
            full((NS, H)), full((G, H)),
        ],
        out_shape=[
            jax.ShapeDtypeStruct((N, H), jnp.float32),
            jax.ShapeDtypeStruct((N, H), jnp.float32),
            jax.ShapeDtypeStruct((NS, H), jnp.float32),
            jax.ShapeDtypeStruct((G, H), jnp.float32),
        ],
        scratch_shapes=[pltpu.VMEM((N2, H), jnp.float32)],
    )(z3, n2s3, emb, x, pxW, pxb, iew1, ieb1, iew2, ieb2,
      inw1, inb1, inw2, inb2, c0w, s2s, s2g)


def _edge_agg(m, src3, dst3, zeros_n):
    mesh = plsc.VectorSubcoreMesh(core_axis_name="c", subcore_axis_name="s")

    @functools.partial(
        pl.kernel,
        mesh=mesh,
        out_type=jax.ShapeDtypeStruct((2, N, H), jnp.float32),
        scratch_types=[
            pltpu.VMEM((EROW,), jnp.int32),
            pltpu.VMEM((EROW,), jnp.int32),
            pltpu.VMEM((EROW, H), jnp.float32),
            pltpu.VMEM_SHARED((NP, H), jnp.float32),
            pltpu.SemaphoreType.DMA,
        ],
    )
    def k(m_hbm, src_hbm, dst_hbm, z_hbm, out_hbm, src_a, dst_a,
          rows_a, acc, sem_a):
        c = lax.axis_index("c")
        s = lax.axis_index("s")
        w = s * 2 + c
        rpt = 624
        ztail = NP - 16 * rpt

        pltpu.sync_copy(z_hbm.at[pl.ds(s * rpt, rpt)],
                        acc.at[pl.ds(s * rpt, rpt)])

        @pl.when(s == 15)
        def _():
            pltpu.sync_copy(z_hbm.at[pl.ds(16 * rpt, ztail)],
                            acc.at[pl.ds(16 * rpt, ztail)])

        plsc.subcore_barrier()

        def body(t, carry):
            pltpu.sync_copy(src_hbm.at[w, t], src_a)
            pltpu.sync_copy(dst_hbm.at[w, t], dst_a)
            pltpu.async_copy(m_hbm.at[src_a], rows_a, sem_a).wait()
            pltpu.sync_copy(rows_a, acc.at[dst_a], add=True)
            return carry

        lax.fori_loop(0, GPW, body, 0)

        plsc.subcore_barrier()
        wtail = N - 16 * rpt
        pltpu.sync_copy(acc.at[pl.ds(s * rpt, rpt)],
                        out_hbm.at[c, pl.ds(s * rpt, rpt)])

        @pl.when(s == 15)
        def _():
            pltpu.sync_copy(acc.at[pl.ds(16 * rpt, wtail)],
                            out_hbm.at[c, pl.ds(16 * rpt, wtail)])

    return k(m, src3, dst3, zeros_n)


def _conv_body(has_next, final, *refs):
    if final:
        (h_ref, p_ref, seg_ref, wih_ref, whh_ref, bih_ref, bhh_ref,
         ew1_ref, eb1_ref, ew2_ref, eb2_ref,
         nw1_ref, nb1_ref, nw2_ref, nb2_ref,
         xf_ref, s2s_ref, s2g_ref, ge0_ref, ge1_ref,
         pw1_ref, pb1_ref, pw2_ref, pb2_ref,
         pred_ref, acc_ref) = refs
    else:
        (h_ref, p_ref, seg_ref, wih_ref, whh_ref, bih_ref, bhh_ref,
         ew1_ref, eb1_ref, ew2_ref, eb2_ref,
         nw1_ref, nb1_ref, nw2_ref, nb2_ref,
         xf_ref, s2s_ref, s2g_ref, cwn_ref,
         hn_ref, mn_ref, ge_ref, acc_ref) = refs

    i = pl.program_id(0)

    @pl.when(i == 0)
    def _():
        acc_ref[...] = jnp.zeros_like(acc_ref)

    h = h_ref[...]
    agg = p_ref[0] + p_ref[1]
    gi = _mm(agg, wih_ref[...], ((1,), (1,))) + bih_ref[...]
    gh = _mm(h, whh_ref[...], ((1,), (1,))) + bhh_ref[...]
    r = _sig(gi[:, :H] + gh[:, :H])
    u = _sig(gi[:, H:2 * H] + gh[:, H:2 * H])
    nn_ = _tanh(gi[:, 2 * H:] + r * gh[:, 2 * H:])
    hn = jnp.maximum((1.0 - u) * nn_ + u * h, 0.0)

    if not final:
        hn_ref[...] = hn
    if has_next:
        mn_ref[...] = _mm(hn, cwn_ref[...], ((1,), (0,)))

    soh = _seg_onehot(seg_ref[0], N2)
    acc_ref[...] += _mm(soh, hn, ((1,), (0,)))

    @pl.when(i == NBLK - 1)
    def _():
        ne = _mlp2(acc_ref[...], ew1_ref[...], eb1_ref[...],
                   ew2_ref[...], eb2_ref[...])
        se = _mm(_seg_onehot(s2s_ref[...], NS), ne, ((1,), (0,)))
        se = _mlp2(se, nw1_ref[...], nb1_ref[...],
                   nw2_ref[...], nb2_ref[...])
        ge = _mm(_seg_onehot(s2g_ref[...], G), se * xf_ref[...],
                 ((1,), (0,)))
        if final:
            embed = ge0_ref[...] + ge1_ref[...] + ge
            hid = jnp.maximum(_mm(embed, pw1_ref[...], ((1,), (0,)))
                              + pb1_ref[...], 0.0)
            pred = _mm(hid, pw2_ref[...], ((1,), (0,))) + pb2_ref[...]
            mx = jnp.max(pred, axis=-1, keepdims=True)
            sh = pred - mx
            lse = jnp.log(jnp.sum(jnp.exp(sh), axis=-1, keepdims=True))
            pred_ref[...] = sh - lse
        else:
            ge_ref[...] = ge


def _conv_layer(h, parts, n2s3, wih, whh, bih, bhh,
                ew1, eb1, ew2, eb2, nw1, nb1, nw2, nb2,
                xf, s2s, s2g, cwn=None, finals=None):
    full = lambda shp: pl.BlockSpec(shp, lambda i: (0,) * len(shp))
    final = finals is not None
    has_next = cwn is not None
    in_specs = [
        pl.BlockSpec((NB, H), lambda i: (i, 0)),
        pl.BlockSpec((2, NB, H), lambda i: (0, i, 0)),
        pl.BlockSpec((1, 1, NB), lambda i: (i, 0, 0)),
        full((3 * H, H)), full((3 * H, H)), full((1, 3 * H)), full((1, 3 * H)),
        full((H, 2 * H)), full((1, 2 * H)), full((2 * H, H)), full((1, H)),
        full((H, 2 * H)), full((1, 2 * H)), full((2 * H, H)), full((1, H)),
        full((NS, H)), full((1, N2)), full((1, NS)),
    ]
    args = [h, parts, n2s3, wih, whh, bih, bhh,
            ew1, eb1, ew2, eb2, nw1, nb1, nw2, nb2, xf, s2s, s2g]
    if final:
        ge0, ge1, pw1, pb1, pw2, pb2 = finals
        in_specs += [full((G, H)), full((G, H)),
                     full((H, H)), full((1, H)), full((H, C)), full((1, C))]
        args += [ge0, ge1, pw1, pb1, pw2, pb2]
        out_specs = [full((G, C))]
        out_shape = [jax.ShapeDtypeStruct((G, C), jnp.float32)]
    else:
        in_specs += [full((H, H))]
        args += [cwn]
        out_specs = [pl.BlockSpec((NB, H), lambda i: (i, 0)),
                     pl.BlockSpec((NB, H), lambda i: (i, 0)),
                     full((G, H))]
        out_shape = [jax.ShapeDtypeStruct((N, H), jnp.float32),
                     jax.ShapeDtypeStruct((N, H), jnp.float32),
                     jax.ShapeDtypeStruct((G, H), jnp.float32)]
    return pl.pallas_call(
        functools.partial(_conv_body, has_next, final),
        grid=(NBLK,),
        in_specs=in_specs,
        out_specs=out_specs,
        out_shape=out_shape,
        scratch_shapes=[pltpu.VMEM((N2, H), jnp.float32)],
    )(*args)


def kernel(z, x, edge_index, batch, node_to_subgraph2, subgraph2_to_subgraph,
           subgraph_to_graph, emb, pxW, pxb, ie_w1, ie_b1, ie_w2, ie_b2,
           in_w1, in_b1, in_w2, in_b2,
           conv0_w, conv0_wih, conv0_whh, conv0_bih, conv0_bhh,
           e0_w1, e0_b1, e0_w2, e0_b2, n0_w1, n0_b1, n0_w2, n0_b2,
           conv1_w, conv1_wih, conv1_whh, conv1_bih, conv1_bhh,
           e1_w1, e1_b1, e1_w2, e1_b2, n1_w1, n1_b1, n1_w2, n1_b2,
           post_w1, post_b1, post_w2, post_b2):
    i32 = jnp.int32
    z3 = z.astype(i32).reshape(NBLK, 1, NB)
    n2s3 = node_to_subgraph2.astype(i32).reshape(NBLK, 1, NB)
    s2s = subgraph2_to_subgraph.astype(i32).reshape(1, N2)
    s2g = subgraph_to_graph.astype(i32).reshape(1, NS)
    srcp = jnp.concatenate(
        [edge_index[0].astype(i32), jnp.zeros((EPAD,), i32)])
    dstp = jnp.concatenate(
        [edge_index[1].astype(i32),
         N + (jnp.arange(EPAD, dtype=i32) % (NP - N))])
    src3 = srcp.reshape(GPW, NW, EROW).transpose(1, 0, 2)
    dst3 = dstp.reshape(GPW, NW, EROW).transpose(1, 0, 2)
    zeros_n = jnp.zeros((NP, H), jnp.float32)
    row = lambda b: b.reshape(1, -1)

    zf, m0, xf, ge0 = _prelude(
        z3, n2s3, emb, x, pxW, row(pxb),
        ie_w1, row(ie_b1), ie_w2, row(ie_b2),
        in_w1, row(in_b1), in_w2, row(in_b2), conv0_w, s2s, s2g)

    parts0 = _edge_agg(m0, src3, dst3, zeros_n)
    h1, m1, ge1 = _conv_layer(
        zf, parts0, n2s3, conv0_wih, conv0_whh, row(conv0_bih),
        row(conv0_bhh), e0_w1, row(e0_b1), e0_w2, row(e0_b2),
        n0_w1, row(n0_b1), n0_w2, row(n0_b2), xf, s2s, s2g, cwn=conv1_w)

    parts1 = _edge_agg(m1, src3, dst3, zeros_n)
    (pred,) = _conv_layer(
        h1, parts1, n2s3, conv1_wih, conv1_whh, row(conv1_bih),
        row(conv1_bhh), e1_w1, row(e1_b1), e1_w2, row(e1_b2),
        n1_w1, row(n1_b1), n1_w2, row(n1_b2), xf, s2s, s2g,
        finals=(ge0, ge1, post_w1, row(post_b1), post_w2, row(post_b2)))
    return pred


# speedup vs baseline: 1.7592x; 1.6118x over previous
import functools

import jax
import jax.numpy as jnp
from jax import lax
from jax.experimental import pallas as pl
from jax.experimental.pallas import tpu as pltpu
from jax.experimental.pallas import tpu_sc as plsc

H = 128
N = 10000
E = 320000
N2 = 2000
NS = 400
G = 16
C = 10

NB = 1000
NBLK = N // NB
EROW = 128
ER = E // EROW
NW = 32
RPW = ER // NW
REM = ER - NW * RPW


def _mm(a, b, dims):
    return lax.dot_general(a, b, dimension_numbers=(dims, ((), ())),
                           preferred_element_type=jnp.float32,
                           precision=lax.Precision.HIGHEST)


def _sig(x):
    return 1.0 / (1.0 + jnp.exp(-x))


def _tanh(x):
    return 2.0 / (1.0 + jnp.exp(-2.0 * x)) - 1.0


def _mlp2(h, w1, b1, w2, b2):
    hid = jnp.maximum(_mm(h, w1, ((1,), (0,))) + b1, 0.0)
    return _mm(hid, w2, ((1,), (0,))) + b2


def _seg_onehot(ids_row, nseg):
    L = ids_row.shape[1]
    return (lax.broadcasted_iota(jnp.int32, (nseg, L), 0) == ids_row
            ).astype(jnp.float32)


def _prelude_body(z_ref, seg_ref, emb_ref, x_ref, pxW_ref, pxb_ref,
                  iew1_ref, ieb1_ref, iew2_ref, ieb2_ref,
                  inw1_ref, inb1_ref, inw2_ref, inb2_ref,
                  c0w_ref, s2s_ref, s2g_ref,
                  zf_ref, m0_ref, xf_ref, ge0_ref, acc_ref):
    i = pl.program_id(0)

    @pl.when(i == 0)
    def _():
        acc_ref[...] = jnp.zeros_like(acc_ref)

    oh = _seg_onehot(z_ref[0], 100)
    zf = jnp.maximum(_mm(oh, emb_ref[...], ((0,), (0,))), 0.0)
    zf_ref[...] = zf
    m0_ref[...] = _mm(zf, c0w_ref[...], ((1,), (0,)))
    soh = _seg_onehot(seg_ref[0], N2)
    acc_ref[...] += _mm(soh, zf, ((1,), (0,)))

    @pl.when(i == NBLK - 1)
    def _():
        xf = jnp.maximum(_mm(x_ref[...], pxW_ref[...], ((1,), (0,)))
                         + pxb_ref[...], 0.0)
        xf_ref[...] = xf
        ne = _mlp2(acc_ref[...], iew1_ref[...], ieb1_ref[...],
                   iew2_ref[...], ieb2_ref[...])
        se = _mm(_seg_onehot(s2s_ref[...], NS), ne, ((1,), (0,)))
        se = _mlp2(se, inw1_ref[...], inb1_ref[...],
                   inw2_ref[...], inb2_ref[...])
        ge0_ref[...] = _mm(_seg_onehot(s2g_ref[...], G), se * xf,
                           ((1,), (0,)))


def _prelude(z3, n2s3, emb, x, pxW, pxb, iew1, ieb1, iew2, ieb2,
             inw1, inb1, inw2, inb2, c0w, s2s, s2g):
    full = lambda shp: pl.BlockSpec(shp, lambda i: (0,) * len(shp))
    return pl.pallas_call(
        _prelude_body,
        grid=(NBLK,),
        in_specs=[
            pl.BlockSpec((1, 1, NB), lambda i: (i, 0, 0)),
            pl.BlockSpec((1, 1, NB), lambda i: (i, 0, 0)),
            full((100, H)), full((NS, H)), full((H, H)), full((1, H)),
            full((H, 2 * H)), full((1, 2 * H)), full((2 * H, H)), full((1, H)),
            full((H, 2 * H)), full((1, 2 * H)), full((2 * H, H)), full((1, H)),
            full((H, H)), full((1, N2)), full((1, NS)),
        ],
        out_specs=[
            pl.BlockSpec((NB, H), lambda i: (i, 0)),
            pl.BlockSpec((NB, H), lambda i: (i, 0)),
            full((NS, H)), full((G, H)),
        ],
        out_shape=[
            jax.ShapeDtypeStruct((N, H), jnp.float32),
            jax.ShapeDtypeStruct((N, H), jnp.float32),
            jax.ShapeDtypeStruct((NS, H), jnp.float32),
            jax.ShapeDtypeStruct((G, H), jnp.float32),
        ],
        scratch_shapes=[pltpu.VMEM((N2, H), jnp.float32)],
    )(z3, n2s3, emb, x, pxW, pxb, iew1, ieb1, iew2, ieb2,
      inw1, inb1, inw2, inb2, c0w, s2s, s2g)


def _edge_agg(m, src2d, dst2d, zeros_n):
    mesh = plsc.VectorSubcoreMesh(core_axis_name="c", subcore_axis_name="s")

    @functools.partial(
        pl.kernel,
        mesh=mesh,
        out_type=jax.ShapeDtypeStruct((2, N, H), jnp.float32),
        scratch_types=[
            pltpu.VMEM((EROW,), jnp.int32),
            pltpu.VMEM((EROW,), jnp.int32),
            pltpu.VMEM((EROW, H), jnp.float32),
            pltpu.VMEM_SHARED((N, H), jnp.float32),
            pltpu.SemaphoreType.DMA,
        ],
    )
    def k(m_hbm, src_hbm, dst_hbm, z_hbm, out_hbm, src_v, dst_v, rows_v,
          acc, sem):
        c = lax.axis_index("c")
        s = lax.axis_index("s")
        w = s * 2 + c
        rpt = 624
        tail = N - 16 * rpt

        pltpu.sync_copy(z_hbm.at[pl.ds(s * rpt, rpt)],
                        acc.at[pl.ds(s * rpt, rpt)])

        @pl.when(s == 15)
        def _():
            pltpu.sync_copy(z_hbm.at[pl.ds(16 * rpt, tail)],
                            acc.at[pl.ds(16 * rpt, tail)])

        plsc.subcore_barrier()

        def do_group(r):
            pltpu.sync_copy(src_hbm.at[r], src_v)
            pltpu.sync_copy(dst_hbm.at[r], dst_v)
            pltpu.async_copy(m_hbm.at[src_v], rows_v, sem).wait()
            pltpu.sync_copy(rows_v, acc.at[dst_v], add=True)

        def body(j, carry):
            do_group(w * RPW + j)
            return carry

        lax.fori_loop(0, RPW, body, 0)

        @pl.when(w < REM)
        def _():
            do_group(NW * RPW + w)

        plsc.subcore_barrier()
        pltpu.sync_copy(acc.at[pl.ds(s * rpt, rpt)],
                        out_hbm.at[c, pl.ds(s * rpt, rpt)])

        @pl.when(s == 15)
        def _():
            pltpu.sync_copy(acc.at[pl.ds(16 * rpt, tail)],
                            out_hbm.at[c, pl.ds(16 * rpt, tail)])

    return k(m, src2d, dst2d, zeros_n)


def _conv_body(has_next, final, *refs):
    if final:
        (h_ref, p_ref, seg_ref, wih_ref, whh_ref, bih_ref, bhh_ref,
         ew1_ref, eb1_ref, ew2_ref, eb2_ref,
         nw1_ref, nb1_ref, nw2_ref, nb2_ref,
         xf_ref, s2s_ref, s2g_ref, ge0_ref, ge1_ref,
         pw1_ref, pb1_ref, pw2_ref, pb2_ref,
         pred_ref, acc_ref) = refs
    else:
        (h_ref, p_ref, seg_ref, wih_ref, whh_ref, bih_ref, bhh_ref,
         ew1_ref, eb1_ref, ew2_ref, eb2_ref,
         nw1_ref, nb1_ref, nw2_ref, nb2_ref,
         xf_ref, s2s_ref, s2g_ref, cwn_ref,
         hn_ref, mn_ref, ge_ref, acc_ref) = refs

    i = pl.program_id(0)

    @pl.when(i == 0)
    def _():
        acc_ref[...] = jnp.zeros_like(acc_ref)

    h = h_ref[...]
    agg = p_ref[0] + p_ref[1]
    gi = _mm(agg, wih_ref[...], ((1,), (1,))) + bih_ref[...]
    gh = _mm(h, whh_ref[...], ((1,), (1,))) + bhh_ref[...]
    r = _sig(gi[:, :H] + gh[:, :H])
    u = _sig(gi[:, H:2 * H] + gh[:, H:2 * H])
    nn_ = _tanh(gi[:, 2 * H:] + r * gh[:, 2 * H:])
    hn = jnp.maximum((1.0 - u) * nn_ + u * h, 0.0)

    if not final:
        hn_ref[...] = hn
    if has_next:
        mn_ref[...] = _mm(hn, cwn_ref[...], ((1,), (0,)))

    soh = _seg_onehot(seg_ref[0], N2)
    acc_ref[...] += _mm(soh, hn, ((1,), (0,)))

    @pl.when(i == NBLK - 1)
    def _():
        ne = _mlp2(acc_ref[...], ew1_ref[...], eb1_ref[...],
                   ew2_ref[...], eb2_ref[...])
        se = _mm(_seg_onehot(s2s_ref[...], NS), ne, ((1,), (0,)))
        se = _mlp2(se, nw1_ref[...], nb1_ref[...],
                   nw2_ref[...], nb2_ref[...])
        ge = _mm(_seg_onehot(s2g_ref[...], G), se * xf_ref[...],
                 ((1,), (0,)))
        if final:
            embed = ge0_ref[...] + ge1_ref[...] + ge
            hid = jnp.maximum(_mm(embed, pw1_ref[...], ((1,), (0,)))
                              + pb1_ref[...], 0.0)
            pred = _mm(hid, pw2_ref[...], ((1,), (0,))) + pb2_ref[...]
            mx = jnp.max(pred, axis=-1, keepdims=True)
            sh = pred - mx
            lse = jnp.log(jnp.sum(jnp.exp(sh), axis=-1, keepdims=True))
            pred_ref[...] = sh - lse
        else:
            ge_ref[...] = ge


def _conv_layer(h, parts, n2s3, wih, whh, bih, bhh,
                ew1, eb1, ew2, eb2, nw1, nb1, nw2, nb2,
                xf, s2s, s2g, cwn=None, finals=None):
    full = lambda shp: pl.BlockSpec(shp, lambda i: (0,) * len(shp))
    final = finals is not None
    has_next = cwn is not None
    in_specs = [
        pl.BlockSpec((NB, H), lambda i: (i, 0)),
        pl.BlockSpec((2, NB, H), lambda i: (0, i, 0)),
        pl.BlockSpec((1, 1, NB), lambda i: (i, 0, 0)),
        full((3 * H, H)), full((3 * H, H)), full((1, 3 * H)), full((1, 3 * H)),
        full((H, 2 * H)), full((1, 2 * H)), full((2 * H, H)), full((1, H)),
        full((H, 2 * H)), full((1, 2 * H)), full((2 * H, H)), full((1, H)),
        full((NS, H)), full((1, N2)), full((1, NS)),
    ]
    args = [h, parts, n2s3, wih, whh, bih, bhh,
            ew1, eb1, ew2, eb2, nw1, nb1, nw2, nb2, xf, s2s, s2g]
    if final:
        ge0, ge1, pw1, pb1, pw2, pb2 = finals
        in_specs += [full((G, H)), full((G, H)),
                     full((H, H)), full((1, H)), full((H, C)), full((1, C))]
        args += [ge0, ge1, pw1, pb1, pw2, pb2]
        out_specs = [full((G, C))]
        out_shape = [jax.ShapeDtypeStruct((G, C), jnp.float32)]
    else:
        in_specs += [full((H, H))]
        args += [cwn]
        out_specs = [pl.BlockSpec((NB, H), lambda i: (i, 0)),
                     pl.BlockSpec((NB, H), lambda i: (i, 0)),
                     full((G, H))]
        out_shape = [jax.ShapeDtypeStruct((N, H), jnp.float32),
                     jax.ShapeDtypeStruct((N, H), jnp.float32),
                     jax.ShapeDtypeStruct((G, H), jnp.float32)]
    return pl.pallas_call(
        functools.partial(_conv_body, has_next, final),
        grid=(NBLK,),
        in_specs=in_specs,
        out_specs=out_specs,
        out_shape=out_shape,
        scratch_shapes=[pltpu.VMEM((N2, H), jnp.float32)],
    )(*args)


def kernel(z, x, edge_index, batch, node_to_subgraph2, subgraph2_to_subgraph,
           subgraph_to_graph, emb, pxW, pxb, ie_w1, ie_b1, ie_w2, ie_b2,
           in_w1, in_b1, in_w2, in_b2,
           conv0_w, conv0_wih, conv0_whh, conv0_bih, conv0_bhh,
           e0_w1, e0_b1, e0_w2, e0_b2, n0_w1, n0_b1, n0_w2, n0_b2,
           conv1_w, conv1_wih, conv1_whh, conv1_bih, conv1_bhh,
           e1_w1, e1_b1, e1_w2, e1_b2, n1_w1, n1_b1, n1_w2, n1_b2,
           post_w1, post_b1, post_w2, post_b2):
    i32 = jnp.int32
    z3 = z.astype(i32).reshape(NBLK, 1, NB)
    n2s3 = node_to_subgraph2.astype(i32).reshape(NBLK, 1, NB)
    s2s = subgraph2_to_subgraph.astype(i32).reshape(1, N2)
    s2g = subgraph_to_graph.astype(i32).reshape(1, NS)
    src2d = edge_index[0].astype(i32).reshape(ER, EROW)
    dst2d = edge_index[1].astype(i32).reshape(ER, EROW)
    zeros_n = jnp.zeros((N, H), jnp.float32)
    row = lambda b: b.reshape(1, -1)

    zf, m0, xf, ge0 = _prelude(
        z3, n2s3, emb, x, pxW, row(pxb),
        ie_w1, row(ie_b1), ie_w2, row(ie_b2),
        in_w1, row(in_b1), in_w2, row(in_b2), conv0_w, s2s, s2g)

    parts0 = _edge_agg(m0, src2d, dst2d, zeros_n)
    h1, m1, ge1 = _conv_layer(
        zf, parts0, n2s3, conv0_wih, conv0_whh, row(conv0_bih),
        row(conv0_bhh), e0_w1, row(e0_b1), e0_w2, row(e0_b2),
        n0_w1, row(n0_b1), n0_w2, row(n0_b2), xf, s2s, s2g, cwn=conv1_w)

    parts1 = _edge_agg(m1, src2d, dst2d, zeros_n)
    (pred,) = _conv_layer(
        h1, parts1, n2s3, conv1_wih, conv1_whh, row(conv1_bih),
        row(conv1_bhh), e1_w1, row(e1_b1), e1_w2, row(e1_b2),
        n1_w1, row(n1_b1), n1_w2, row(n1_b2), xf, s2s, s2g,
        finals=(ge0, ge1, post_w1, row(post_b1), post_w2, row(post_b2)))
    return pred


# speedup vs baseline: 2.3459x; 1.3334x over previous
import functools

import jax
import jax.numpy as jnp
from jax import lax
from jax.experimental import pallas as pl
from jax.experimental.pallas import tpu as pltpu
from jax.experimental.pallas import tpu_sc as plsc

H = 128
N = 10000
E = 320000
N2 = 2000
NS = 400
G = 16
C = 10

NB = 1000
NBLK = N // NB
EROW = 128
ER = E // EROW
NW = 32
RPW = ER // NW
REM = ER - NW * RPW


def _dot(a, b, dims):
    return lax.dot_general(a, b, dimension_numbers=(dims, ((), ())),
                           preferred_element_type=jnp.float32)


def _split(x):
    hi = x.astype(jnp.bfloat16)
    lo = (x - hi.astype(jnp.float32)).astype(jnp.bfloat16)
    return hi, lo


def _mm(a, b, dims):
    ah, al = _split(a)
    bh, bl = _split(b)
    return _dot(ah, bh, dims) + (_dot(ah, bl, dims) + _dot(al, bh, dims))


def _mm1(a_exact, b, dims):
    ah = a_exact.astype(jnp.bfloat16)
    bh, bl = _split(b)
    return _dot(ah, bh, dims) + _dot(ah, bl, dims)


def _sig(x):
    return 1.0 / (1.0 + jnp.exp(-x))


def _tanh(x):
    return 2.0 / (1.0 + jnp.exp(-2.0 * x)) - 1.0


def _mlp2(h, w1, b1, w2, b2):
    hid = jnp.maximum(_mm(h, w1, ((1,), (0,))) + b1, 0.0)
    return _mm(hid, w2, ((1,), (0,))) + b2


def _seg_onehot(ids_row, nseg):
    L = ids_row.shape[1]
    return (lax.broadcasted_iota(jnp.int32, (nseg, L), 0) == ids_row
            ).astype(jnp.float32)


def _prelude_body(z_ref, seg_ref, emb_ref, x_ref, pxW_ref, pxb_ref,
                  iew1_ref, ieb1_ref, iew2_ref, ieb2_ref,
                  inw1_ref, inb1_ref, inw2_ref, inb2_ref,
                  c0w_ref, s2s_ref, s2g_ref,
                  zf_ref, m0_ref, xf_ref, ge0_ref, acc_ref):
    i = pl.program_id(0)

    @pl.when(i == 0)
    def _():
        acc_ref[...] = jnp.zeros_like(acc_ref)

    oh = _seg_onehot(z_ref[0], 100)
    zf = jnp.maximum(_mm1(oh, emb_ref[...], ((0,), (0,))), 0.0)
    zf_ref[...] = zf
    m0_ref[...] = _mm(zf, c0w_ref[...], ((1,), (0,)))
    soh = _seg_onehot(seg_ref[0], N2)
    acc_ref[...] += _mm1(soh, zf, ((1,), (0,)))

    @pl.when(i == NBLK - 1)
    def _():
        xf = jnp.maximum(_mm(x_ref[...], pxW_ref[...], ((1,), (0,)))
                         + pxb_ref[...], 0.0)
        xf_ref[...] = xf
        ne = _mlp2(acc_ref[...], iew1_ref[...], ieb1_ref[...],
                   iew2_ref[...], ieb2_ref[...])
        se = _mm1(_seg_onehot(s2s_ref[...], NS), ne, ((1,), (0,)))
        se = _mlp2(se, inw1_ref[...], inb1_ref[...],
                   inw2_ref[...], inb2_ref[...])
        ge0_ref[...] = _mm1(_seg_onehot(s2g_ref[...], G), se * xf,
                           ((1,), (0,)))


def _prelude(z3, n2s3, emb, x, pxW, pxb, iew1, ieb1, iew2, ieb2,
             inw1, inb1, inw2, inb2, c0w, s2s, s2g):
    full = lambda shp: pl.BlockSpec(shp, lambda i: (0,) * len(shp))
    return pl.pallas_call(
        _prelude_body,
        grid=(NBLK,),
        in_specs=[
            pl.BlockSpec((1, 1, NB), lambda i: (i, 0, 0)),
            pl.BlockSpec((1, 1, NB), lambda i: (i, 0, 0)),
            full((100, H)), full((NS, H)), full((H, H)), full((1, H)),
            full((H, 2 * H)), full((1, 2 * H)), full((2 * H, H)), full((1, H)),
            full((H, 2 * H)), full((1, 2 * H)), full((2 * H, H)), full((1, H)),
            full((H, H)), full((1, N2)), full((1, NS)),
        ],
        out_specs=[
            pl.BlockSpec((NB, H), lambda i: (i, 0)),
            pl.BlockSpec((NB, H), lambda i: (i, 0)),
            full((NS, H)), full((G, H)),
        ],
        out_shape=[
            jax.ShapeDtypeStruct((N, H), jnp.float32),
            jax.ShapeDtypeStruct((N, H), jnp.float32),
            jax.ShapeDtypeStruct((NS, H), jnp.float32),
            jax.ShapeDtypeStruct((G, H), jnp.float32),
        ],
        scratch_shapes=[pltpu.VMEM((N2, H), jnp.float32)],
    )(z3, n2s3, emb, x, pxW, pxb, iew1, ieb1, iew2, ieb2,
      inw1, inb1, inw2, inb2, c0w, s2s, s2g)


def _edge_agg(m, src2d, dst2d, zeros_n):
    mesh = plsc.VectorSubcoreMesh(core_axis_name="c", subcore_axis_name="s")

    @functools.partial(
        pl.kernel,
        mesh=mesh,
        out_type=jax.ShapeDtypeStruct((2, N, H), jnp.float32),
        scratch_types=[
            pltpu.VMEM((EROW,), jnp.int32),
            pltpu.VMEM((EROW,), jnp.int32),
            pltpu.VMEM((EROW, H), jnp.float32),
            pltpu.VMEM_SHARED((N, H), jnp.float32),
            pltpu.SemaphoreType.DMA,
        ],
    )
    def k(m_hbm, src_hbm, dst_hbm, z_hbm, out_hbm, src_v, dst_v, rows_v,
          acc, sem):
        c = lax.axis_index("c")
        s = lax.axis_index("s")
        w = s * 2 + c
        rpt = 624
        tail = N - 16 * rpt

        pltpu.sync_copy(z_hbm.at[pl.ds(s * rpt, rpt)],
                        acc.at[pl.ds(s * rpt, rpt)])

        @pl.when(s == 15)
        def _():
            pltpu.sync_copy(z_hbm.at[pl.ds(16 * rpt, tail)],
                            acc.at[pl.ds(16 * rpt, tail)])

        plsc.subcore_barrier()

        def do_group(r):
            pltpu.sync_copy(src_hbm.at[r], src_v)
            pltpu.sync_copy(dst_hbm.at[r], dst_v)
            pltpu.async_copy(m_hbm.at[src_v], rows_v, sem).wait()
            pltpu.sync_copy(rows_v, acc.at[dst_v], add=True)

        def body(j, carry):
            do_group(w * RPW + j)
            return carry

        lax.fori_loop(0, RPW, body, 0)

        @pl.when(w < REM)
        def _():
            do_group(NW * RPW + w)

        plsc.subcore_barrier()
        pltpu.sync_copy(acc.at[pl.ds(s * rpt, rpt)],
                        out_hbm.at[c, pl.ds(s * rpt, rpt)])

        @pl.when(s == 15)
        def _():
            pltpu.sync_copy(acc.at[pl.ds(16 * rpt, tail)],
                            out_hbm.at[c, pl.ds(16 * rpt, tail)])

    return k(m, src2d, dst2d, zeros_n)


def _conv_body(has_next, final, *refs):
    if final:
        (h_ref, p_ref, seg_ref, wih_ref, whh_ref, bih_ref, bhh_ref,
         ew1_ref, eb1_ref, ew2_ref, eb2_ref,
         nw1_ref, nb1_ref, nw2_ref, nb2_ref,
         xf_ref, s2s_ref, s2g_ref, ge0_ref, ge1_ref,
         pw1_ref, pb1_ref, pw2_ref, pb2_ref,
         pred_ref, acc_ref) = refs
    else:
        (h_ref, p_ref, seg_ref, wih_ref, whh_ref, bih_ref, bhh_ref,
         ew1_ref, eb1_ref, ew2_ref, eb2_ref,
         nw1_ref, nb1_ref, nw2_ref, nb2_ref,
         xf_ref, s2s_ref, s2g_ref, cwn_ref,
         hn_ref, mn_ref, ge_ref, acc_ref) = refs

    i = pl.program_id(0)

    @pl.when(i == 0)
    def _():
        acc_ref[...] = jnp.zeros_like(acc_ref)

    h = h_ref[...]
    agg = p_ref[0] + p_ref[1]
    gi = _mm(agg, wih_ref[...], ((1,), (1,))) + bih_ref[...]
    gh = _mm(h, whh_ref[...], ((1,), (1,))) + bhh_ref[...]
    r = _sig(gi[:, :H] + gh[:, :H])
    u = _sig(gi[:, H:2 * H] + gh[:, H:2 * H])
    nn_ = _tanh(gi[:, 2 * H:] + r * gh[:, 2 * H:])
    hn = jnp.maximum((1.0 - u) * nn_ + u * h, 0.0)

    if not final:
        hn_ref[...] = hn
    if has_next:
        mn_ref[...] = _mm(hn, cwn_ref[...], ((1,), (0,)))

    soh = _seg_onehot(seg_ref[0], N2)
    acc_ref[...] += _mm1(soh, hn, ((1,), (0,)))

    @pl.when(i == NBLK - 1)
    def _():
        ne = _mlp2(acc_ref[...], ew1_ref[...], eb1_ref[...],
                   ew2_ref[...], eb2_ref[...])
        se = _mm1(_seg_onehot(s2s_ref[...], NS), ne, ((1,), (0,)))
        se = _mlp2(se, nw1_ref[...], nb1_ref[...],
                   nw2_ref[...], nb2_ref[...])
        ge = _mm1(_seg_onehot(s2g_ref[...], G), se * xf_ref[...],
                 ((1,), (0,)))
        if final:
            embed = ge0_ref[...] + ge1_ref[...] + ge
            hid = jnp.maximum(_mm(embed, pw1_ref[...], ((1,), (0,)))
                              + pb1_ref[...], 0.0)
            pred = _mm(hid, pw2_ref[...], ((1,), (0,))) + pb2_ref[...]
            mx = jnp.max(pred, axis=-1, keepdims=True)
            sh = pred - mx
            lse = jnp.log(jnp.sum(jnp.exp(sh), axis=-1, keepdims=True))
            pred_ref[...] = sh - lse
        else:
            ge_ref[...] = ge


def _conv_layer(h, parts, n2s3, wih, whh, bih, bhh,
                ew1, eb1, ew2, eb2, nw1, nb1, nw2, nb2,
                xf, s2s, s2g, cwn=None, finals=None):
    full = lambda shp: pl.BlockSpec(shp, lambda i: (0,) * len(shp))
    final = finals is not None
    has_next = cwn is not None
    in_specs = [
        pl.BlockSpec((NB, H), lambda i: (i, 0)),
        pl.BlockSpec((2, NB, H), lambda i: (0, i, 0)),
        pl.BlockSpec((1, 1, NB), lambda i: (i, 0, 0)),
        full((3 * H, H)), full((3 * H, H)), full((1, 3 * H)), full((1, 3 * H)),
        full((H, 2 * H)), full((1, 2 * H)), full((2 * H, H)), full((1, H)),
        full((H, 2 * H)), full((1, 2 * H)), full((2 * H, H)), full((1, H)),
        full((NS, H)), full((1, N2)), full((1, NS)),
    ]
    args = [h, parts, n2s3, wih, whh, bih, bhh,
            ew1, eb1, ew2, eb2, nw1, nb1, nw2, nb2, xf, s2s, s2g]
    if final:
        ge0, ge1, pw1, pb1, pw2, pb2 = finals
        in_specs += [full((G, H)), full((G, H)),
                     full((H, H)), full((1, H)), full((H, C)), full((1, C))]
        args += [ge0, ge1, pw1, pb1, pw2, pb2]
        out_specs = [full((G, C))]
        out_shape = [jax.ShapeDtypeStruct((G, C), jnp.float32)]
    else:
        in_specs += [full((H, H))]
        args += [cwn]
        out_specs = [pl.BlockSpec((NB, H), lambda i: (i, 0)),
                     pl.BlockSpec((NB, H), lambda i: (i, 0)),
                     full((G, H))]
        out_shape = [jax.ShapeDtypeStruct((N, H), jnp.float32),
                     jax.ShapeDtypeStruct((N, H), jnp.float32),
                     jax.ShapeDtypeStruct((G, H), jnp.float32)]
    return pl.pallas_call(
        functools.partial(_conv_body, has_next, final),
        grid=(NBLK,),
        in_specs=in_specs,
        out_specs=out_specs,
        out_shape=out_shape,
        scratch_shapes=[pltpu.VMEM((N2, H), jnp.float32)],
    )(*args)


def kernel(z, x, edge_index, batch, node_to_subgraph2, subgraph2_to_subgraph,
           subgraph_to_graph, emb, pxW, pxb, ie_w1, ie_b1, ie_w2, ie_b2,
           in_w1, in_b1, in_w2, in_b2,
           conv0_w, conv0_wih, conv0_whh, conv0_bih, conv0_bhh,
           e0_w1, e0_b1, e0_w2, e0_b2, n0_w1, n0_b1, n0_w2, n0_b2,
           conv1_w, conv1_wih, conv1_whh, conv1_bih, conv1_bhh,
           e1_w1, e1_b1, e1_w2, e1_b2, n1_w1, n1_b1, n1_w2, n1_b2,
           post_w1, post_b1, post_w2, post_b2):
    i32 = jnp.int32
    z3 = z.astype(i32).reshape(NBLK, 1, NB)
    n2s3 = node_to_subgraph2.astype(i32).reshape(NBLK, 1, NB)
    s2s = subgraph2_to_subgraph.astype(i32).reshape(1, N2)
    s2g = subgraph_to_graph.astype(i32).reshape(1, NS)
    src2d = edge_index[0].astype(i32).reshape(ER, EROW)
    dst2d = edge_index[1].astype(i32).reshape(ER, EROW)
    zeros_n = jnp.zeros((N, H), jnp.float32)
    row = lambda b: b.reshape(1, -1)

    zf, m0, xf, ge0 = _prelude(
        z3, n2s3, emb, x, pxW, row(pxb),
        ie_w1, row(ie_b1), ie_w2, row(ie_b2),
        in_w1, row(in_b1), in_w2, row(in_b2), conv0_w, s2s, s2g)

    parts0 = _edge_agg(m0, src2d, dst2d, zeros_n)
    h1, m1, ge1 = _conv_layer(
        zf, parts0, n2s3, conv0_wih, conv0_whh, row(conv0_bih),
        row(conv0_bhh), e0_w1, row(e0_b1), e0_w2, row(e0_b2),
        n0_w1, row(n0_b1), n0_w2, row(n0_b2), xf, s2s, s2g, cwn=conv1_w)

    parts1 = _edge_agg(m1, src2d, dst2d, zeros_n)
    (pred,) = _conv_layer(
        h1, parts1, n2s3, conv1_wih, conv1_whh, row(conv1_bih),
        row(conv1_bhh), e1_w1, row(e1_b1), e1_w2, row(e1_b2),
        n1_w1, row(n1_b1), n1_w2, row(n1_b2), xf, s2s, s2g,
        finals=(ge0, ge1, post_w1, row(post_b1), post_w2, row(post_b2)))
    return pred


# speedup vs baseline: 2.9518x; 1.2583x over previous
import functools

import jax
import jax.numpy as jnp
from jax import lax
from jax.experimental import pallas as pl
from jax.experimental.pallas import tpu as pltpu
from jax.experimental.pallas import tpu_sc as plsc

H = 128
N = 10000
E = 320000
N2 = 2000
NS = 400
G = 16
C = 10

NB = 1000
NBLK = N // NB
EROW = 128
ER = E // EROW
NW = 32
RPW = ER // NW
REM = ER - NW * RPW


def _dot(a, b, dims):
    return lax.dot_general(a, b, dimension_numbers=(dims, ((), ())),
                           preferred_element_type=jnp.float32)


def _split(x):
    hi = x.astype(jnp.bfloat16)
    lo = (x - hi.astype(jnp.float32)).astype(jnp.bfloat16)
    return hi, lo


def _mm(a, b, dims):
    ah, al = _split(a)
    bh, bl = _split(b)
    return _dot(ah, bh, dims) + (_dot(ah, bl, dims) + _dot(al, bh, dims))


def _mm1(a_exact, b, dims):
    ah = a_exact.astype(jnp.bfloat16)
    bh, bl = _split(b)
    return _dot(ah, bh, dims) + _dot(ah, bl, dims)


def _sig(x):
    return 1.0 / (1.0 + jnp.exp(-x))


def _tanh(x):
    return 2.0 / (1.0 + jnp.exp(-2.0 * x)) - 1.0


def _mlp2(h, w1, b1, w2, b2):
    hid = jnp.maximum(_mm(h, w1, ((1,), (0,))) + b1, 0.0)
    return _mm(hid, w2, ((1,), (0,))) + b2


def _seg_onehot(ids_row, nseg):
    L = ids_row.shape[1]
    return (lax.broadcasted_iota(jnp.int32, (nseg, L), 0) == ids_row
            ).astype(jnp.float32)


def _prelude_body(z_ref, seg_ref, emb_ref, x_ref, pxW_ref, pxb_ref,
                  iew1_ref, ieb1_ref, iew2_ref, ieb2_ref,
                  inw1_ref, inb1_ref, inw2_ref, inb2_ref,
                  c0w_ref, s2s_ref, s2g_ref,
                  zf_ref, m0_ref, xf_ref, ge0_ref, acc_ref):
    i = pl.program_id(0)

    @pl.when(i == 0)
    def _():
        acc_ref[...] = jnp.zeros_like(acc_ref)

    oh = _seg_onehot(z_ref[0], 100)
    zf = jnp.maximum(_mm1(oh, emb_ref[...], ((0,), (0,))), 0.0)
    zf_ref[...] = zf
    m0_ref[...] = _mm(zf, c0w_ref[...], ((1,), (0,)))
    soh = _seg_onehot(seg_ref[0], N2)
    acc_ref[...] += _mm1(soh, zf, ((1,), (0,)))

    @pl.when(i == NBLK - 1)
    def _():
        xf = jnp.maximum(_mm(x_ref[...], pxW_ref[...], ((1,), (0,)))
                         + pxb_ref[...], 0.0)
        xf_ref[...] = xf
        ne = _mlp2(acc_ref[...], iew1_ref[...], ieb1_ref[...],
                   iew2_ref[...], ieb2_ref[...])
        se = _mm1(_seg_onehot(s2s_ref[...], NS), ne, ((1,), (0,)))
        se = _mlp2(se, inw1_ref[...], inb1_ref[...],
                   inw2_ref[...], inb2_ref[...])
        ge0_ref[...] = _mm1(_seg_onehot(s2g_ref[...], G), se * xf,
                           ((1,), (0,)))


def _prelude(z3, n2s3, emb, x, pxW, pxb, iew1, ieb1, iew2, ieb2,
             inw1, inb1, inw2, inb2, c0w, s2s, s2g):
    full = lambda shp: pl.BlockSpec(shp, lambda i: (0,) * len(shp))
    return pl.pallas_call(
        _prelude_body,
        grid=(NBLK,),
        in_specs=[
            pl.BlockSpec((1, 1, NB), lambda i: (i, 0, 0)),
            pl.BlockSpec((1, 1, NB), lambda i: (i, 0, 0)),
            full((100, H)), full((NS, H)), full((H, H)), full((1, H)),
            full((H, 2 * H)), full((1, 2 * H)), full((2 * H, H)), full((1, H)),
            full((H, 2 * H)), full((1, 2 * H)), full((2 * H, H)), full((1, H)),
            full((H, H)), full((1, N2)), full((1, NS)),
        ],
        out_specs=[
            pl.BlockSpec((NB, H), lambda i: (i, 0)),
            pl.BlockSpec((NB, H), lambda i: (i, 0)),
            full((NS, H)), full((G, H)),
        ],
        out_shape=[
            jax.ShapeDtypeStruct((N, H), jnp.float32),
            jax.ShapeDtypeStruct((N, H), jnp.float32),
            jax.ShapeDtypeStruct((NS, H), jnp.float32),
            jax.ShapeDtypeStruct((G, H), jnp.float32),
        ],
        scratch_shapes=[pltpu.VMEM((N2, H), jnp.float32)],
    )(z3, n2s3, emb, x, pxW, pxb, iew1, ieb1, iew2, ieb2,
      inw1, inb1, inw2, inb2, c0w, s2s, s2g)


def _edge_agg(m, src2d, dst2d, zeros_n):
    mesh = plsc.VectorSubcoreMesh(core_axis_name="c", subcore_axis_name="s")

    @functools.partial(
        pl.kernel,
        mesh=mesh,
        out_type=jax.ShapeDtypeStruct((2, N, H), jnp.float32),
        scratch_types=[
            pltpu.VMEM((EROW,), jnp.int32),
            pltpu.VMEM((EROW,), jnp.int32),
            pltpu.VMEM((EROW,), jnp.int32),
            pltpu.VMEM((EROW,), jnp.int32),
            pltpu.VMEM((EROW, H), jnp.float32),
            pltpu.VMEM_SHARED((N, H), jnp.float32),
            pltpu.SemaphoreType.DMA,
        ],
    )
    def k(m_hbm, src_hbm, dst_hbm, z_hbm, out_hbm, src_a, dst_a, src_b,
          dst_b, rows_v, acc, sem):
        c = lax.axis_index("c")
        s = lax.axis_index("s")
        w = s * 2 + c
        rpt = 624
        tail = N - 16 * rpt

        pltpu.sync_copy(z_hbm.at[pl.ds(s * rpt, rpt)],
                        acc.at[pl.ds(s * rpt, rpt)])

        @pl.when(s == 15)
        def _():
            pltpu.sync_copy(z_hbm.at[pl.ds(16 * rpt, tail)],
                            acc.at[pl.ds(16 * rpt, tail)])

        plsc.subcore_barrier()

        def load_idx(r, sv, dv):
            pltpu.sync_copy(src_hbm.at[r], sv)
            pltpu.sync_copy(dst_hbm.at[r], dv)

        base = w * RPW
        last = base + RPW - 1
        load_idx(base, src_a, dst_a)

        def body(t, carry):
            ga = base + 2 * t
            cp = pltpu.async_copy(m_hbm.at[src_a], rows_v, sem)
            load_idx(ga + 1, src_b, dst_b)
            cp.wait()
            pltpu.sync_copy(rows_v, acc.at[dst_a], add=True)
            cp = pltpu.async_copy(m_hbm.at[src_b], rows_v, sem)
            load_idx(jnp.minimum(ga + 2, last), src_a, dst_a)
            cp.wait()
            pltpu.sync_copy(rows_v, acc.at[dst_b], add=True)
            return carry

        lax.fori_loop(0, RPW // 2, body, 0)

        @pl.when(w < REM)
        def _():
            load_idx(NW * RPW + w, src_a, dst_a)
            pltpu.async_copy(m_hbm.at[src_a], rows_v, sem).wait()
            pltpu.sync_copy(rows_v, acc.at[dst_a], add=True)

        plsc.subcore_barrier()
        pltpu.sync_copy(acc.at[pl.ds(s * rpt, rpt)],
                        out_hbm.at[c, pl.ds(s * rpt, rpt)])

        @pl.when(s == 15)
        def _():
            pltpu.sync_copy(acc.at[pl.ds(16 * rpt, tail)],
                            out_hbm.at[c, pl.ds(16 * rpt, tail)])

    return k(m, src2d, dst2d, zeros_n)


def _conv_body(has_next, final, *refs):
    if final:
        (h_ref, p_ref, seg_ref, wih_ref, whh_ref, bih_ref, bhh_ref,
         ew1_ref, eb1_ref, ew2_ref, eb2_ref,
         nw1_ref, nb1_ref, nw2_ref, nb2_ref,
         xf_ref, s2s_ref, s2g_ref, ge0_ref, ge1_ref,
         pw1_ref, pb1_ref, pw2_ref, pb2_ref,
         pred_ref, acc_ref) = refs
    else:
        (h_ref, p_ref, seg_ref, wih_ref, whh_ref, bih_ref, bhh_ref,
         ew1_ref, eb1_ref, ew2_ref, eb2_ref,
         nw1_ref, nb1_ref, nw2_ref, nb2_ref,
         xf_ref, s2s_ref, s2g_ref, cwn_ref,
         hn_ref, mn_ref, ge_ref, acc_ref) = refs

    i = pl.program_id(0)

    @pl.when(i == 0)
    def _():
        acc_ref[...] = jnp.zeros_like(acc_ref)

    h = h_ref[...]
    agg = p_ref[0] + p_ref[1]
    gi = _mm(agg, wih_ref[...], ((1,), (1,))) + bih_ref[...]
    gh = _mm(h, whh_ref[...], ((1,), (1,))) + bhh_ref[...]
    r = _sig(gi[:, :H] + gh[:, :H])
    u = _sig(gi[:, H:2 * H] + gh[:, H:2 * H])
    nn_ = _tanh(gi[:, 2 * H:] + r * gh[:, 2 * H:])
    hn = jnp.maximum((1.0 - u) * nn_ + u * h, 0.0)

    if not final:
        hn_ref[...] = hn
    if has_next:
        mn_ref[...] = _mm(hn, cwn_ref[...], ((1,), (0,)))

    soh = _seg_onehot(seg_ref[0], N2)
    acc_ref[...] += _mm1(soh, hn, ((1,), (0,)))

    @pl.when(i == NBLK - 1)
    def _():
        ne = _mlp2(acc_ref[...], ew1_ref[...], eb1_ref[...],
                   ew2_ref[...], eb2_ref[...])
        se = _mm1(_seg_onehot(s2s_ref[...], NS), ne, ((1,), (0,)))
        se = _mlp2(se, nw1_ref[...], nb1_ref[...],
                   nw2_ref[...], nb2_ref[...])
        ge = _mm1(_seg_onehot(s2g_ref[...], G), se * xf_ref[...],
                 ((1,), (0,)))
        if final:
            embed = ge0_ref[...] + ge1_ref[...] + ge
            hid = jnp.maximum(_mm(embed, pw1_ref[...], ((1,), (0,)))
                              + pb1_ref[...], 0.0)
            pred = _mm(hid, pw2_ref[...], ((1,), (0,))) + pb2_ref[...]
            mx = jnp.max(pred, axis=-1, keepdims=True)
            sh = pred - mx
            lse = jnp.log(jnp.sum(jnp.exp(sh), axis=-1, keepdims=True))
            pred_ref[...] = sh - lse
        else:
            ge_ref[...] = ge


def _conv_layer(h, parts, n2s3, wih, whh, bih, bhh,
                ew1, eb1, ew2, eb2, nw1, nb1, nw2, nb2,
                xf, s2s, s2g, cwn=None, finals=None):
    full = lambda shp: pl.BlockSpec(shp, lambda i: (0,) * len(shp))
    final = finals is not None
    has_next = cwn is not None
    in_specs = [
        pl.BlockSpec((NB, H), lambda i: (i, 0)),
        pl.BlockSpec((2, NB, H), lambda i: (0, i, 0)),
        pl.BlockSpec((1, 1, NB), lambda i: (i, 0, 0)),
        full((3 * H, H)), full((3 * H, H)), full((1, 3 * H)), full((1, 3 * H)),
        full((H, 2 * H)), full((1, 2 * H)), full((2 * H, H)), full((1, H)),
        full((H, 2 * H)), full((1, 2 * H)), full((2 * H, H)), full((1, H)),
        full((NS, H)), full((1, N2)), full((1, NS)),
    ]
    args = [h, parts, n2s3, wih, whh, bih, bhh,
            ew1, eb1, ew2, eb2, nw1, nb1, nw2, nb2, xf, s2s, s2g]
    if final:
        ge0, ge1, pw1, pb1, pw2, pb2 = finals
        in_specs += [full((G, H)), full((G, H)),
                     full((H, H)), full((1, H)), full((H, C)), full((1, C))]
        args += [ge0, ge1, pw1, pb1, pw2, pb2]
        out_specs = [full((G, C))]
        out_shape = [jax.ShapeDtypeStruct((G, C), jnp.float32)]
    else:
        in_specs += [full((H, H))]
        args += [cwn]
        out_specs = [pl.BlockSpec((NB, H), lambda i: (i, 0)),
                     pl.BlockSpec((NB, H), lambda i: (i, 0)),
                     full((G, H))]
        out_shape = [jax.ShapeDtypeStruct((N, H), jnp.float32),
                     jax.ShapeDtypeStruct((N, H), jnp.float32),
                     jax.ShapeDtypeStruct((G, H), jnp.float32)]
    return pl.pallas_call(
        functools.partial(_conv_body, has_next, final),
        grid=(NBLK,),
        in_specs=in_specs,
        out_specs=out_specs,
        out_shape=out_shape,
        scratch_shapes=[pltpu.VMEM((N2, H), jnp.float32)],
    )(*args)


def kernel(z, x, edge_index, batch, node_to_subgraph2, subgraph2_to_subgraph,
           subgraph_to_graph, emb, pxW, pxb, ie_w1, ie_b1, ie_w2, ie_b2,
           in_w1, in_b1, in_w2, in_b2,
           conv0_w, conv0_wih, conv0_whh, conv0_bih, conv0_bhh,
           e0_w1, e0_b1, e0_w2, e0_b2, n0_w1, n0_b1, n0_w2, n0_b2,
           conv1_w, conv1_wih, conv1_whh, conv1_bih, conv1_bhh,
           e1_w1, e1_b1, e1_w2, e1_b2, n1_w1, n1_b1, n1_w2, n1_b2,
           post_w1, post_b1, post_w2, post_b2):
    i32 = jnp.int32
    z3 = z.astype(i32).reshape(NBLK, 1, NB)
    n2s3 = node_to_subgraph2.astype(i32).reshape(NBLK, 1, NB)
    s2s = subgraph2_to_subgraph.astype(i32).reshape(1, N2)
    s2g = subgraph_to_graph.astype(i32).reshape(1, NS)
    src2d = edge_index[0].astype(i32).reshape(ER, EROW)
    dst2d = edge_index[1].astype(i32).reshape(ER, EROW)
    zeros_n = jnp.zeros((N, H), jnp.float32)
    row = lambda b: b.reshape(1, -1)

    zf, m0, xf, ge0 = _prelude(
        z3, n2s3, emb, x, pxW, row(pxb),
        ie_w1, row(ie_b1), ie_w2, row(ie_b2),
        in_w1, row(in_b1), in_w2, row(in_b2), conv0_w, s2s, s2g)

    parts0 = _edge_agg(m0, src2d, dst2d, zeros_n)
    h1, m1, ge1 = _conv_layer(
        zf, parts0, n2s3, conv0_wih, conv0_whh, row(conv0_bih),
        row(conv0_bhh), e0_w1, row(e0_b1), e0_w2, row(e0_b2),
        n0_w1, row(n0_b1), n0_w2, row(n0_b2), xf, s2s, s2g, cwn=conv1_w)

    parts1 = _edge_agg(m1, src2d, dst2d, zeros_n)
    (pred,) = _conv_layer(
        h1, parts1, n2s3, conv1_wih, conv1_whh, row(conv1_bih),
        row(conv1_bhh), e1_w1, row(e1_b1), e1_w2, row(e1_b2),
        n1_w1, row(n1_b1), n1_w2, row(n1_b2), xf, s2s, s2g,
        finals=(ge0, ge1, post_w1, row(post_b1), post_w2, row(post_b2)))
    return pred


# speedup vs baseline: 3.1046x; 1.0518x over previous
import functools

import jax
import jax.numpy as jnp
from jax import lax
from jax.experimental import pallas as pl
from jax.experimental.pallas import tpu as pltpu
from jax.experimental.pallas import tpu_sc as plsc

H = 128
N = 10000
E = 320000
N2 = 2000
NS = 400
G = 16
C = 10

NB = 1000
NBLK = N // NB
EROW = 128
ER = E // EROW
NW = 32
RPW = ER // NW
REM = ER - NW * RPW


def _dot(a, b, dims):
    return lax.dot_general(a, b, dimension_numbers=(dims, ((), ())),
                           preferred_element_type=jnp.float32)


def _split(x):
    hi = x.astype(jnp.bfloat16)
    lo = (x - hi.astype(jnp.float32)).astype(jnp.bfloat16)
    return hi, lo


def _mm(a, b, dims):
    ah, al = _split(a)
    bh, bl = _split(b)
    return _dot(ah, bh, dims) + (_dot(ah, bl, dims) + _dot(al, bh, dims))


def _mm1(a_exact, b, dims):
    ah = a_exact.astype(jnp.bfloat16)
    bh, bl = _split(b)
    return _dot(ah, bh, dims) + _dot(ah, bl, dims)


def _sig(x):
    return 1.0 / (1.0 + jnp.exp(-x))


def _tanh(x):
    return 2.0 / (1.0 + jnp.exp(-2.0 * x)) - 1.0


def _mlp2(h, w1, b1, w2, b2):
    hid = jnp.maximum(_mm(h, w1, ((1,), (0,))) + b1, 0.0)
    return _mm(hid, w2, ((1,), (0,))) + b2


def _seg_onehot(ids_row, nseg):
    L = ids_row.shape[1]
    return (lax.broadcasted_iota(jnp.int32, (nseg, L), 0) == ids_row
            ).astype(jnp.float32)


def _prelude_body(z_ref, seg_ref, emb_ref, x_ref, pxW_ref, pxb_ref,
                  iew1_ref, ieb1_ref, iew2_ref, ieb2_ref,
                  inw1_ref, inb1_ref, inw2_ref, inb2_ref,
                  c0w_ref, s2s_ref, s2g_ref,
                  zf_ref, m0_ref, xf_ref, ge0_ref, acc_ref):
    i = pl.program_id(0)

    @pl.when(i == 0)
    def _():
        acc_ref[...] = jnp.zeros_like(acc_ref)

    oh = _seg_onehot(z_ref[0], 100)
    zf = jnp.maximum(_mm1(oh, emb_ref[...], ((0,), (0,))), 0.0)
    zf_ref[...] = zf
    m0_ref[...] = _mm(zf, c0w_ref[...], ((1,), (0,)))
    soh = _seg_onehot(seg_ref[0], N2)
    acc_ref[...] += _mm1(soh, zf, ((1,), (0,)))

    @pl.when(i == NBLK - 1)
    def _():
        xf = jnp.maximum(_mm(x_ref[...], pxW_ref[...], ((1,), (0,)))
                         + pxb_ref[...], 0.0)
        xf_ref[...] = xf
        ne = _mlp2(acc_ref[...], iew1_ref[...], ieb1_ref[...],
                   iew2_ref[...], ieb2_ref[...])
        se = _mm1(_seg_onehot(s2s_ref[...], NS), ne, ((1,), (0,)))
        se = _mlp2(se, inw1_ref[...], inb1_ref[...],
                   inw2_ref[...], inb2_ref[...])
        ge0_ref[...] = _mm1(_seg_onehot(s2g_ref[...], G), se * xf,
                           ((1,), (0,)))


def _prelude(z3, n2s3, emb, x, pxW, pxb, iew1, ieb1, iew2, ieb2,
             inw1, inb1, inw2, inb2, c0w, s2s, s2g):
    full = lambda shp: pl.BlockSpec(shp, lambda i: (0,) * len(shp))
    return pl.pallas_call(
        _prelude_body,
        grid=(NBLK,),
        in_specs=[
            pl.BlockSpec((1, 1, NB), lambda i: (i, 0, 0)),
            pl.BlockSpec((1, 1, NB), lambda i: (i, 0, 0)),
            full((100, H)), full((NS, H)), full((H, H)), full((1, H)),
            full((H, 2 * H)), full((1, 2 * H)), full((2 * H, H)), full((1, H)),
            full((H, 2 * H)), full((1, 2 * H)), full((2 * H, H)), full((1, H)),
            full((H, H)), full((1, N2)), full((1, NS)),
        ],
        out_specs=[
            pl.BlockSpec((NB, H), lambda i: (i, 0)),
            pl.BlockSpec((NB, H), lambda i: (i, 0)),
            full((NS, H)), full((G, H)),
        ],
        out_shape=[
            jax.ShapeDtypeStruct((N, H), jnp.float32),
            jax.ShapeDtypeStruct((N, H), jnp.float32),
            jax.ShapeDtypeStruct((NS, H), jnp.float32),
            jax.ShapeDtypeStruct((G, H), jnp.float32),
        ],
        scratch_shapes=[pltpu.VMEM((N2, H), jnp.float32)],
    )(z3, n2s3, emb, x, pxW, pxb, iew1, ieb1, iew2, ieb2,
      inw1, inb1, inw2, inb2, c0w, s2s, s2g)


def _edge_agg(m, src2d, dst2d, zeros_n):
    mesh = plsc.VectorSubcoreMesh(core_axis_name="c", subcore_axis_name="s")

    @functools.partial(
        pl.kernel,
        mesh=mesh,
        out_type=jax.ShapeDtypeStruct((2, N, H), jnp.float32),
        scratch_types=[
            pltpu.VMEM((EROW,), jnp.int32),
            pltpu.VMEM((EROW,), jnp.int32),
            pltpu.VMEM((EROW,), jnp.int32),
            pltpu.VMEM((EROW,), jnp.int32),
            pltpu.VMEM((EROW, H), jnp.float32),
            pltpu.VMEM((EROW, H), jnp.float32),
            pltpu.VMEM_SHARED((N, H), jnp.float32),
            pltpu.SemaphoreType.DMA,
            pltpu.SemaphoreType.DMA,
        ],
    )
    def k(m_hbm, src_hbm, dst_hbm, z_hbm, out_hbm, src_a, dst_a, src_b,
          dst_b, rows_a, rows_b, acc, sem, sem_b):
        c = lax.axis_index("c")
        s = lax.axis_index("s")
        w = s * 2 + c
        rpt = 624
        tail = N - 16 * rpt

        pltpu.sync_copy(z_hbm.at[pl.ds(s * rpt, rpt)],
                        acc.at[pl.ds(s * rpt, rpt)])

        @pl.when(s == 15)
        def _():
            pltpu.sync_copy(z_hbm.at[pl.ds(16 * rpt, tail)],
                            acc.at[pl.ds(16 * rpt, tail)])

        plsc.subcore_barrier()

        def load_idx(r, sv, dv):
            pltpu.sync_copy(src_hbm.at[r], sv)
            pltpu.sync_copy(dst_hbm.at[r], dv)

        base = w * RPW
        last = base + RPW - 1
        load_idx(base, src_a, dst_a)

        def body(t, carry):
            ga = base + 2 * t
            cpa = pltpu.async_copy(m_hbm.at[src_a], rows_a, sem)
            load_idx(ga + 1, src_b, dst_b)
            cpa.wait()
            cpb = pltpu.async_copy(m_hbm.at[src_b], rows_b, sem_b)
            pltpu.sync_copy(rows_a, acc.at[dst_a], add=True)
            load_idx(jnp.minimum(ga + 2, last), src_a, dst_a)
            cpb.wait()
            pltpu.sync_copy(rows_b, acc.at[dst_b], add=True)
            return carry

        lax.fori_loop(0, RPW // 2, body, 0)

        @pl.when(w < REM)
        def _():
            load_idx(NW * RPW + w, src_a, dst_a)
            pltpu.async_copy(m_hbm.at[src_a], rows_a, sem).wait()
            pltpu.sync_copy(rows_a, acc.at[dst_a], add=True)

        plsc.subcore_barrier()
        pltpu.sync_copy(acc.at[pl.ds(s * rpt, rpt)],
                        out_hbm.at[c, pl.ds(s * rpt, rpt)])

        @pl.when(s == 15)
        def _():
            pltpu.sync_copy(acc.at[pl.ds(16 * rpt, tail)],
                            out_hbm.at[c, pl.ds(16 * rpt, tail)])

    return k(m, src2d, dst2d, zeros_n)


def _conv_body(has_next, final, *refs):
    if final:
        (h_ref, p_ref, seg_ref, wih_ref, whh_ref, bih_ref, bhh_ref,
         ew1_ref, eb1_ref, ew2_ref, eb2_ref,
         nw1_ref, nb1_ref, nw2_ref, nb2_ref,
         xf_ref, s2s_ref, s2g_ref, ge0_ref, ge1_ref,
         pw1_ref, pb1_ref, pw2_ref, pb2_ref,
         pred_ref, acc_ref) = refs
    else:
        (h_ref, p_ref, seg_ref, wih_ref, whh_ref, bih_ref, bhh_ref,
         ew1_ref, eb1_ref, ew2_ref, eb2_ref,
         nw1_ref, nb1_ref, nw2_ref, nb2_ref,
         xf_ref, s2s_ref, s2g_ref, cwn_ref,
         hn_ref, mn_ref, ge_ref, acc_ref) = refs

    i = pl.program_id(0)

    @pl.when(i == 0)
    def _():
        acc_ref[...] = jnp.zeros_like(acc_ref)

    h = h_ref[...]
    agg = p_ref[0] + p_ref[1]
    gi = _mm(agg, wih_ref[...], ((1,), (1,))) + bih_ref[...]
    gh = _mm(h, whh_ref[...], ((1,), (1,))) + bhh_ref[...]
    r = _sig(gi[:, :H] + gh[:, :H])
    u = _sig(gi[:, H:2 * H] + gh[:, H:2 * H])
    nn_ = _tanh(gi[:, 2 * H:] + r * gh[:, 2 * H:])
    hn = jnp.maximum((1.0 - u) * nn_ + u * h, 0.0)

    if not final:
        hn_ref[...] = hn
    if has_next:
        mn_ref[...] = _mm(hn, cwn_ref[...], ((1,), (0,)))

    soh = _seg_onehot(seg_ref[0], N2)
    acc_ref[...] += _mm1(soh, hn, ((1,), (0,)))

    @pl.when(i == NBLK - 1)
    def _():
        ne = _mlp2(acc_ref[...], ew1_ref[...], eb1_ref[...],
                   ew2_ref[...], eb2_ref[...])
        se = _mm1(_seg_onehot(s2s_ref[...], NS), ne, ((1,), (0,)))
        se = _mlp2(se, nw1_ref[...], nb1_ref[...],
                   nw2_ref[...], nb2_ref[...])
        ge = _mm1(_seg_onehot(s2g_ref[...], G), se * xf_ref[...],
                 ((1,), (0,)))
        if final:
            embed = ge0_ref[...] + ge1_ref[...] + ge
            hid = jnp.maximum(_mm(embed, pw1_ref[...], ((1,), (0,)))
                              + pb1_ref[...], 0.0)
            pred = _mm(hid, pw2_ref[...], ((1,), (0,))) + pb2_ref[...]
            mx = jnp.max(pred, axis=-1, keepdims=True)
            sh = pred - mx
            lse = jnp.log(jnp.sum(jnp.exp(sh), axis=-1, keepdims=True))
            pred_ref[...] = sh - lse
        else:
            ge_ref[...] = ge


def _conv_layer(h, parts, n2s3, wih, whh, bih, bhh,
                ew1, eb1, ew2, eb2, nw1, nb1, nw2, nb2,
                xf, s2s, s2g, cwn=None, finals=None):
    full = lambda shp: pl.BlockSpec(shp, lambda i: (0,) * len(shp))
    final = finals is not None
    has_next = cwn is not None
    in_specs = [
        pl.BlockSpec((NB, H), lambda i: (i, 0)),
        pl.BlockSpec((2, NB, H), lambda i: (0, i, 0)),
        pl.BlockSpec((1, 1, NB), lambda i: (i, 0, 0)),
        full((3 * H, H)), full((3 * H, H)), full((1, 3 * H)), full((1, 3 * H)),
        full((H, 2 * H)), full((1, 2 * H)), full((2 * H, H)), full((1, H)),
        full((H, 2 * H)), full((1, 2 * H)), full((2 * H, H)), full((1, H)),
        full((NS, H)), full((1, N2)), full((1, NS)),
    ]
    args = [h, parts, n2s3, wih, whh, bih, bhh,
            ew1, eb1, ew2, eb2, nw1, nb1, nw2, nb2, xf, s2s, s2g]
    if final:
        ge0, ge1, pw1, pb1, pw2, pb2 = finals
        in_specs += [full((G, H)), full((G, H)),
                     full((H, H)), full((1, H)), full((H, C)), full((1, C))]
        args += [ge0, ge1, pw1, pb1, pw2, pb2]
        out_specs = [full((G, C))]
        out_shape = [jax.ShapeDtypeStruct((G, C), jnp.float32)]
    else:
        in_specs += [full((H, H))]
        args += [cwn]
        out_specs = [pl.BlockSpec((NB, H), lambda i: (i, 0)),
                     pl.BlockSpec((NB, H), lambda i: (i, 0)),
                     full((G, H))]
        out_shape = [jax.ShapeDtypeStruct((N, H), jnp.float32),
                     jax.ShapeDtypeStruct((N, H), jnp.float32),
                     jax.ShapeDtypeStruct((G, H), jnp.float32)]
    return pl.pallas_call(
        functools.partial(_conv_body, has_next, final),
        grid=(NBLK,),
        in_specs=in_specs,
        out_specs=out_specs,
        out_shape=out_shape,
        scratch_shapes=[pltpu.VMEM((N2, H), jnp.float32)],
    )(*args)


def kernel(z, x, edge_index, batch, node_to_subgraph2, subgraph2_to_subgraph,
           subgraph_to_graph, emb, pxW, pxb, ie_w1, ie_b1, ie_w2, ie_b2,
           in_w1, in_b1, in_w2, in_b2,
           conv0_w, conv0_wih, conv0_whh, conv0_bih, conv0_bhh,
           e0_w1, e0_b1, e0_w2, e0_b2, n0_w1, n0_b1, n0_w2, n0_b2,
           conv1_w, conv1_wih, conv1_whh, conv1_bih, conv1_bhh,
           e1_w1, e1_b1, e1_w2, e1_b2, n1_w1, n1_b1, n1_w2, n1_b2,
           post_w1, post_b1, post_w2, post_b2):
    i32 = jnp.int32
    z3 = z.astype(i32).reshape(NBLK, 1, NB)
    n2s3 = node_to_subgraph2.astype(i32).reshape(NBLK, 1, NB)
    s2s = subgraph2_to_subgraph.astype(i32).reshape(1, N2)
    s2g = subgraph_to_graph.astype(i32).reshape(1, NS)
    src2d = edge_index[0].astype(i32).reshape(ER, EROW)
    dst2d = edge_index[1].astype(i32).reshape(ER, EROW)
    zeros_n = jnp.zeros((N, H), jnp.float32)
    row = lambda b: b.reshape(1, -1)

    zf, m0, xf, ge0 = _prelude(
        z3, n2s3, emb, x, pxW, row(pxb),
        ie_w1, row(ie_b1), ie_w2, row(ie_b2),
        in_w1, row(in_b1), in_w2, row(in_b2), conv0_w, s2s, s2g)

    parts0 = _edge_agg(m0, src2d, dst2d, zeros_n)
    h1, m1, ge1 = _conv_layer(
        zf, parts0, n2s3, conv0_wih, conv0_whh, row(conv0_bih),
        row(conv0_bhh), e0_w1, row(e0_b1), e0_w2, row(e0_b2),
        n0_w1, row(n0_b1), n0_w2, row(n0_b2), xf, s2s, s2g, cwn=conv1_w)

    parts1 = _edge_agg(m1, src2d, dst2d, zeros_n)
    (pred,) = _conv_layer(
        h1, parts1, n2s3, conv1_wih, conv1_whh, row(conv1_bih),
        row(conv1_bhh), e1_w1, row(e1_b1), e1_w2, row(e1_b2),
        n1_w1, row(n1_b1), n1_w2, row(n1_b2), xf, s2s, s2g,
        finals=(ge0, ge1, post_w1, row(post_b1), post_w2, row(post_b2)))
    return pred


# speedup vs baseline: 3.3486x; 1.0786x over previous
import functools

import jax
import jax.numpy as jnp
from jax import lax
from jax.experimental import pallas as pl
from jax.experimental.pallas import tpu as pltpu
from jax.experimental.pallas import tpu_sc as plsc

H = 128
N = 10000
E = 320000
N2 = 2000
NS = 400
G = 16
C = 10

NB = 1000
NBLK = N // NB
EROW = 128
ER = E // EROW
NW = 32
RPW = ER // NW
REM = ER - NW * RPW


def _dot(a, b, dims):
    return lax.dot_general(a, b, dimension_numbers=(dims, ((), ())),
                           preferred_element_type=jnp.float32)


def _split(x):
    hi = x.astype(jnp.bfloat16)
    lo = (x - hi.astype(jnp.float32)).astype(jnp.bfloat16)
    return hi, lo


def _mm(a, b, dims):
    ah, al = _split(a)
    bh, bl = _split(b)
    return _dot(ah, bh, dims) + (_dot(ah, bl, dims) + _dot(al, bh, dims))


def _mm1(a_exact, b, dims):
    ah = a_exact.astype(jnp.bfloat16)
    bh, bl = _split(b)
    return _dot(ah, bh, dims) + _dot(ah, bl, dims)


def _sig(x):
    return 1.0 / (1.0 + jnp.exp(-x))


def _tanh(x):
    return 2.0 / (1.0 + jnp.exp(-2.0 * x)) - 1.0


def _mlp2(h, w1, b1, w2, b2):
    hid = jnp.maximum(_mm(h, w1, ((1,), (0,))) + b1, 0.0)
    return _mm(hid, w2, ((1,), (0,))) + b2


def _seg_onehot(ids_row, nseg):
    L = ids_row.shape[1]
    return (lax.broadcasted_iota(jnp.int32, (nseg, L), 0) == ids_row
            ).astype(jnp.float32)


def _seg_accum(acc_ref, ids, data, nseg, chunk=256):
    L = ids.shape[1]
    mn = jnp.min(ids)
    mx = jnp.max(ids)
    dh, dl = _split(data)

    for k0 in range(0, nseg, chunk):
        sz = min(chunk, nseg - k0)

        def _do(k0=k0, sz=sz):
            ohk = (lax.broadcasted_iota(jnp.int32, (sz, L), 0) + k0 == ids
                   ).astype(jnp.bfloat16)
            acc_ref[k0:k0 + sz, :] += (_dot(ohk, dh, ((1,), (0,)))
                                       + _dot(ohk, dl, ((1,), (0,))))

        pl.when((mx >= k0) & (mn < k0 + sz))(_do)


def _prelude_body(z_ref, seg_ref, emb_ref, x_ref, pxW_ref, pxb_ref,
                  iew1_ref, ieb1_ref, iew2_ref, ieb2_ref,
                  inw1_ref, inb1_ref, inw2_ref, inb2_ref,
                  c0w_ref, s2s_ref, s2g_ref,
                  zf_ref, m0_ref, xf_ref, ge0_ref, acc_ref):
    i = pl.program_id(0)

    @pl.when(i == 0)
    def _():
        acc_ref[...] = jnp.zeros_like(acc_ref)

    oh = _seg_onehot(z_ref[0], 100)
    zf = jnp.maximum(_mm1(oh, emb_ref[...], ((0,), (0,))), 0.0)
    zf_ref[...] = zf
    m0_ref[...] = _mm(zf, c0w_ref[...], ((1,), (0,)))
    _seg_accum(acc_ref, seg_ref[0], zf, N2)

    @pl.when(i == NBLK - 1)
    def _():
        xf = jnp.maximum(_mm(x_ref[...], pxW_ref[...], ((1,), (0,)))
                         + pxb_ref[...], 0.0)
        xf_ref[...] = xf
        ne = _mlp2(acc_ref[...], iew1_ref[...], ieb1_ref[...],
                   iew2_ref[...], ieb2_ref[...])
        se = _mm1(_seg_onehot(s2s_ref[...], NS), ne, ((1,), (0,)))
        se = _mlp2(se, inw1_ref[...], inb1_ref[...],
                   inw2_ref[...], inb2_ref[...])
        ge0_ref[...] = _mm1(_seg_onehot(s2g_ref[...], G), se * xf,
                           ((1,), (0,)))


def _prelude(z3, n2s3, emb, x, pxW, pxb, iew1, ieb1, iew2, ieb2,
             inw1, inb1, inw2, inb2, c0w, s2s, s2g):
    full = lambda shp: pl.BlockSpec(shp, lambda i: (0,) * len(shp))
    return pl.pallas_call(
        _prelude_body,
        grid=(NBLK,),
        in_specs=[
            pl.BlockSpec((1, 1, NB), lambda i: (i, 0, 0)),
            pl.BlockSpec((1, 1, NB), lambda i: (i, 0, 0)),
            full((100, H)), full((NS, H)), full((H, H)), full((1, H)),
            full((H, 2 * H)), full((1, 2 * H)), full((2 * H, H)), full((1, H)),
            full((H, 2 * H)), full((1, 2 * H)), full((2 * H, H)), full((1, H)),
            full((H, H)), full((1, N2)), full((1, NS)),
        ],
        out_specs=[
            pl.BlockSpec((NB, H), lambda i: (i, 0)),
            pl.BlockSpec((NB, H), lambda i: (i, 0)),
            full((NS, H)), full((G, H)),
        ],
        out_shape=[
            jax.ShapeDtypeStruct((N, H), jnp.float32),
            jax.ShapeDtypeStruct((N, H), jnp.float32),
            jax.ShapeDtypeStruct((NS, H), jnp.float32),
            jax.ShapeDtypeStruct((G, H), jnp.float32),
        ],
        scratch_shapes=[pltpu.VMEM((N2, H), jnp.float32)],
    )(z3, n2s3, emb, x, pxW, pxb, iew1, ieb1, iew2, ieb2,
      inw1, inb1, inw2, inb2, c0w, s2s, s2g)


def _edge_agg(m, src2d, dst2d, zeros_n):
    mesh = plsc.VectorSubcoreMesh(core_axis_name="c", subcore_axis_name="s")

    @functools.partial(
        pl.kernel,
        mesh=mesh,
        out_type=jax.ShapeDtypeStruct((2, N, H), jnp.float32),
        scratch_types=[
            pltpu.VMEM((EROW,), jnp.int32),
            pltpu.VMEM((EROW,), jnp.int32),
            pltpu.VMEM((EROW,), jnp.int32),
            pltpu.VMEM((EROW,), jnp.int32),
            pltpu.VMEM((EROW, H), jnp.float32),
            pltpu.VMEM((EROW, H), jnp.float32),
            pltpu.VMEM_SHARED((N, H), jnp.float32),
            pltpu.SemaphoreType.DMA,
            pltpu.SemaphoreType.DMA,
        ],
    )
    def k(m_hbm, src_hbm, dst_hbm, z_hbm, out_hbm, src_a, dst_a, src_b,
          dst_b, rows_a, rows_b, acc, sem, sem_b):
        c = lax.axis_index("c")
        s = lax.axis_index("s")
        w = s * 2 + c
        rpt = 624
        tail = N - 16 * rpt

        pltpu.sync_copy(z_hbm.at[pl.ds(s * rpt, rpt)],
                        acc.at[pl.ds(s * rpt, rpt)])

        @pl.when(s == 15)
        def _():
            pltpu.sync_copy(z_hbm.at[pl.ds(16 * rpt, tail)],
                            acc.at[pl.ds(16 * rpt, tail)])

        plsc.subcore_barrier()

        def load_idx(r, sv, dv):
            pltpu.sync_copy(src_hbm.at[r], sv)
            pltpu.sync_copy(dst_hbm.at[r], dv)

        base = w * RPW
        last = base + RPW - 1
        load_idx(base, src_a, dst_a)

        def body(t, carry):
            ga = base + 2 * t
            cpa = pltpu.async_copy(m_hbm.at[src_a], rows_a, sem)
            load_idx(ga + 1, src_b, dst_b)
            cpa.wait()
            cpb = pltpu.async_copy(m_hbm.at[src_b], rows_b, sem_b)
            pltpu.sync_copy(rows_a, acc.at[dst_a], add=True)
            load_idx(jnp.minimum(ga + 2, last), src_a, dst_a)
            cpb.wait()
            pltpu.sync_copy(rows_b, acc.at[dst_b], add=True)
            return carry

        lax.fori_loop(0, RPW // 2, body, 0)

        @pl.when(w < REM)
        def _():
            load_idx(NW * RPW + w, src_a, dst_a)
            pltpu.async_copy(m_hbm.at[src_a], rows_a, sem).wait()
            pltpu.sync_copy(rows_a, acc.at[dst_a], add=True)

        plsc.subcore_barrier()
        pltpu.sync_copy(acc.at[pl.ds(s * rpt, rpt)],
                        out_hbm.at[c, pl.ds(s * rpt, rpt)])

        @pl.when(s == 15)
        def _():
            pltpu.sync_copy(acc.at[pl.ds(16 * rpt, tail)],
                            out_hbm.at[c, pl.ds(16 * rpt, tail)])

    return k(m, src2d, dst2d, zeros_n)


def _conv_body(has_next, final, *refs):
    if final:
        (h_ref, p_ref, seg_ref, wih_ref, whh_ref, bih_ref, bhh_ref,
         ew1_ref, eb1_ref, ew2_ref, eb2_ref,
         nw1_ref, nb1_ref, nw2_ref, nb2_ref,
         xf_ref, s2s_ref, s2g_ref, ge0_ref, ge1_ref,
         pw1_ref, pb1_ref, pw2_ref, pb2_ref,
         pred_ref, acc_ref) = refs
    else:
        (h_ref, p_ref, seg_ref, wih_ref, whh_ref, bih_ref, bhh_ref,
         ew1_ref, eb1_ref, ew2_ref, eb2_ref,
         nw1_ref, nb1_ref, nw2_ref, nb2_ref,
         xf_ref, s2s_ref, s2g_ref, cwn_ref,
         hn_ref, mn_ref, ge_ref, acc_ref) = refs

    i = pl.program_id(0)

    @pl.when(i == 0)
    def _():
        acc_ref[...] = jnp.zeros_like(acc_ref)

    h = h_ref[...]
    agg = p_ref[0] + p_ref[1]
    gi = _mm(agg, wih_ref[...], ((1,), (1,))) + bih_ref[...]
    gh = _mm(h, whh_ref[...], ((1,), (1,))) + bhh_ref[...]
    r = _sig(gi[:, :H] + gh[:, :H])
    u = _sig(gi[:, H:2 * H] + gh[:, H:2 * H])
    nn_ = _tanh(gi[:, 2 * H:] + r * gh[:, 2 * H:])
    hn = jnp.maximum((1.0 - u) * nn_ + u * h, 0.0)

    if not final:
        hn_ref[...] = hn
    if has_next:
        mn_ref[...] = _mm(hn, cwn_ref[...], ((1,), (0,)))

    _seg_accum(acc_ref, seg_ref[0], hn, N2)

    @pl.when(i == NBLK - 1)
    def _():
        ne = _mlp2(acc_ref[...], ew1_ref[...], eb1_ref[...],
                   ew2_ref[...], eb2_ref[...])
        se = _mm1(_seg_onehot(s2s_ref[...], NS), ne, ((1,), (0,)))
        se = _mlp2(se, nw1_ref[...], nb1_ref[...],
                   nw2_ref[...], nb2_ref[...])
        ge = _mm1(_seg_onehot(s2g_ref[...], G), se * xf_ref[...],
                 ((1,), (0,)))
        if final:
            embed = ge0_ref[...] + ge1_ref[...] + ge
            hid = jnp.maximum(_mm(embed, pw1_ref[...], ((1,), (0,)))
                              + pb1_ref[...], 0.0)
            pred = _mm(hid, pw2_ref[...], ((1,), (0,))) + pb2_ref[...]
            mx = jnp.max(pred, axis=-1, keepdims=True)
            sh = pred - mx
            lse = jnp.log(jnp.sum(jnp.exp(sh), axis=-1, keepdims=True))
            pred_ref[...] = sh - lse
        else:
            ge_ref[...] = ge


def _conv_layer(h, parts, n2s3, wih, whh, bih, bhh,
                ew1, eb1, ew2, eb2, nw1, nb1, nw2, nb2,
                xf, s2s, s2g, cwn=None, finals=None):
    full = lambda shp: pl.BlockSpec(shp, lambda i: (0,) * len(shp))
    final = finals is not None
    has_next = cwn is not None
    in_specs = [
        pl.BlockSpec((NB, H), lambda i: (i, 0)),
        pl.BlockSpec((2, NB, H), lambda i: (0, i, 0)),
        pl.BlockSpec((1, 1, NB), lambda i: (i, 0, 0)),
        full((3 * H, H)), full((3 * H, H)), full((1, 3 * H)), full((1, 3 * H)),
        full((H, 2 * H)), full((1, 2 * H)), full((2 * H, H)), full((1, H)),
        full((H, 2 * H)), full((1, 2 * H)), full((2 * H, H)), full((1, H)),
        full((NS, H)), full((1, N2)), full((1, NS)),
    ]
    args = [h, parts, n2s3, wih, whh, bih, bhh,
            ew1, eb1, ew2, eb2, nw1, nb1, nw2, nb2, xf, s2s, s2g]
    if final:
        ge0, ge1, pw1, pb1, pw2, pb2 = finals
        in_specs += [full((G, H)), full((G, H)),
                     full((H, H)), full((1, H)), full((H, C)), full((1, C))]
        args += [ge0, ge1, pw1, pb1, pw2, pb2]
        out_specs = [full((G, C))]
        out_shape = [jax.ShapeDtypeStruct((G, C), jnp.float32)]
    else:
        in_specs += [full((H, H))]
        args += [cwn]
        out_specs = [pl.BlockSpec((NB, H), lambda i: (i, 0)),
                     pl.BlockSpec((NB, H), lambda i: (i, 0)),
                     full((G, H))]
        out_shape = [jax.ShapeDtypeStruct((N, H), jnp.float32),
                     jax.ShapeDtypeStruct((N, H), jnp.float32),
                     jax.ShapeDtypeStruct((G, H), jnp.float32)]
    return pl.pallas_call(
        functools.partial(_conv_body, has_next, final),
        grid=(NBLK,),
        in_specs=in_specs,
        out_specs=out_specs,
        out_shape=out_shape,
        scratch_shapes=[pltpu.VMEM((N2, H), jnp.float32)],
    )(*args)


def kernel(z, x, edge_index, batch, node_to_subgraph2, subgraph2_to_subgraph,
           subgraph_to_graph, emb, pxW, pxb, ie_w1, ie_b1, ie_w2, ie_b2,
           in_w1, in_b1, in_w2, in_b2,
           conv0_w, conv0_wih, conv0_whh, conv0_bih, conv0_bhh,
           e0_w1, e0_b1, e0_w2, e0_b2, n0_w1, n0_b1, n0_w2, n0_b2,
           conv1_w, conv1_wih, conv1_whh, conv1_bih, conv1_bhh,
           e1_w1, e1_b1, e1_w2, e1_b2, n1_w1, n1_b1, n1_w2, n1_b2,
           post_w1, post_b1, post_w2, post_b2):
    i32 = jnp.int32
    z3 = z.astype(i32).reshape(NBLK, 1, NB)
    n2s3 = node_to_subgraph2.astype(i32).reshape(NBLK, 1, NB)
    s2s = subgraph2_to_subgraph.astype(i32).reshape(1, N2)
    s2g = subgraph_to_graph.astype(i32).reshape(1, NS)
    src2d = edge_index[0].astype(i32).reshape(ER, EROW)
    dst2d = edge_index[1].astype(i32).reshape(ER, EROW)
    zeros_n = jnp.zeros((N, H), jnp.float32)
    row = lambda b: b.reshape(1, -1)

    zf, m0, xf, ge0 = _prelude(
        z3, n2s3, emb, x, pxW, row(pxb),
        ie_w1, row(ie_b1), ie_w2, row(ie_b2),
        in_w1, row(in_b1), in_w2, row(in_b2), conv0_w, s2s, s2g)

    parts0 = _edge_agg(m0, src2d, dst2d, zeros_n)
    h1, m1, ge1 = _conv_layer(
        zf, parts0, n2s3, conv0_wih, conv0_whh, row(conv0_bih),
        row(conv0_bhh), e0_w1, row(e0_b1), e0_w2, row(e0_b2),
        n0_w1, row(n0_b1), n0_w2, row(n0_b2), xf, s2s, s2g, cwn=conv1_w)

    parts1 = _edge_agg(m1, src2d, dst2d, zeros_n)
    (pred,) = _conv_layer(
        h1, parts1, n2s3, conv1_wih, conv1_whh, row(conv1_bih),
        row(conv1_bhh), e1_w1, row(e1_b1), e1_w2, row(e1_b2),
        n1_w1, row(n1_b1), n1_w2, row(n1_b2), xf, s2s, s2g,
        finals=(ge0, ge1, post_w1, row(post_b1), post_w2, row(post_b2)))
    return pred


# speedup vs baseline: 3.5276x; 1.0535x over previous
import functools

import jax
import jax.numpy as jnp
from jax import lax
from jax.experimental import pallas as pl
from jax.experimental.pallas import tpu as pltpu
from jax.experimental.pallas import tpu_sc as plsc

H = 128
N = 10000
E = 320000
N2 = 2000
NS = 400
G = 16
C = 10

NB = 1000
NBLK = N // NB
EROW = 128
ER = E // EROW
NW = 32
RPW = ER // NW
REM = ER - NW * RPW


def _dot(a, b, dims):
    return lax.dot_general(a, b, dimension_numbers=(dims, ((), ())),
                           preferred_element_type=jnp.float32)


def _split(x):
    hi = x.astype(jnp.bfloat16)
    lo = (x - hi.astype(jnp.float32)).astype(jnp.bfloat16)
    return hi, lo


def _mm(a, b, dims):
    ah, al = _split(a)
    bh, bl = _split(b)
    return _dot(ah, bh, dims) + (_dot(ah, bl, dims) + _dot(al, bh, dims))


def _mm1(a_exact, b, dims):
    ah = a_exact.astype(jnp.bfloat16)
    bh, bl = _split(b)
    return _dot(ah, bh, dims) + _dot(ah, bl, dims)


def _sig(x):
    return 1.0 / (1.0 + jnp.exp(-x))


def _tanh(x):
    return 2.0 / (1.0 + jnp.exp(-2.0 * x)) - 1.0


def _mlp2(h, w1, b1, w2, b2):
    hid = jnp.maximum(_mm(h, w1, ((1,), (0,))) + b1, 0.0)
    return _mm(hid, w2, ((1,), (0,))) + b2


def _seg_onehot(ids_row, nseg):
    L = ids_row.shape[1]
    return (lax.broadcasted_iota(jnp.int32, (nseg, L), 0) == ids_row
            ).astype(jnp.float32)


def _seg_accum(acc_ref, ids, data, nseg, chunk=256):
    L = ids.shape[1]
    mn = jnp.min(ids)
    mx = jnp.max(ids)
    dh, dl = _split(data)

    for k0 in range(0, nseg, chunk):
        sz = min(chunk, nseg - k0)

        def _do(k0=k0, sz=sz):
            ohk = (lax.broadcasted_iota(jnp.int32, (sz, L), 0) + k0 == ids
                   ).astype(jnp.bfloat16)
            acc_ref[k0:k0 + sz, :] += (_dot(ohk, dh, ((1,), (0,)))
                                       + _dot(ohk, dl, ((1,), (0,))))

        pl.when((mx >= k0) & (mn < k0 + sz))(_do)


def _prelude_body(z_ref, seg_ref, emb_ref, x_ref, pxW_ref, pxb_ref,
                  iew1_ref, ieb1_ref, iew2_ref, ieb2_ref,
                  inw1_ref, inb1_ref, inw2_ref, inb2_ref,
                  c0w_ref, s2s_ref, s2g_ref,
                  zf_ref, m0_ref, xf_ref, ge0_ref, acc_ref):
    i = pl.program_id(0)

    @pl.when(i == 0)
    def _():
        acc_ref[...] = jnp.zeros_like(acc_ref)

    oh = _seg_onehot(z_ref[0], 100)
    zf = jnp.maximum(_mm1(oh, emb_ref[...], ((0,), (0,))), 0.0)
    zf_ref[...] = zf
    m0_ref[...] = _mm(zf, c0w_ref[...], ((1,), (0,)))
    _seg_accum(acc_ref, seg_ref[0], zf, N2)

    @pl.when(i == NBLK - 1)
    def _():
        xf = jnp.maximum(_mm(x_ref[...], pxW_ref[...], ((1,), (0,)))
                         + pxb_ref[...], 0.0)
        xf_ref[...] = xf
        ne = _mlp2(acc_ref[...], iew1_ref[...], ieb1_ref[...],
                   iew2_ref[...], ieb2_ref[...])
        se = _mm1(_seg_onehot(s2s_ref[...], NS), ne, ((1,), (0,)))
        se = _mlp2(se, inw1_ref[...], inb1_ref[...],
                   inw2_ref[...], inb2_ref[...])
        ge0_ref[...] = _mm1(_seg_onehot(s2g_ref[...], G), se * xf,
                           ((1,), (0,)))


def _prelude(z3, n2s3, emb, x, pxW, pxb, iew1, ieb1, iew2, ieb2,
             inw1, inb1, inw2, inb2, c0w, s2s, s2g):
    full = lambda shp: pl.BlockSpec(shp, lambda i: (0,) * len(shp))
    return pl.pallas_call(
        _prelude_body,
        grid=(NBLK,),
        in_specs=[
            pl.BlockSpec((1, 1, NB), lambda i: (i, 0, 0)),
            pl.BlockSpec((1, 1, NB), lambda i: (i, 0, 0)),
            full((100, H)), full((NS, H)), full((H, H)), full((1, H)),
            full((H, 2 * H)), full((1, 2 * H)), full((2 * H, H)), full((1, H)),
            full((H, 2 * H)), full((1, 2 * H)), full((2 * H, H)), full((1, H)),
            full((H, H)), full((1, N2)), full((1, NS)),
        ],
        out_specs=[
            pl.BlockSpec((NB, H), lambda i: (i, 0)),
            pl.BlockSpec((NB, H), lambda i: (i, 0)),
            full((NS, H)), full((G, H)),
        ],
        out_shape=[
            jax.ShapeDtypeStruct((N, H), jnp.float32),
            jax.ShapeDtypeStruct((N, H), jnp.float32),
            jax.ShapeDtypeStruct((NS, H), jnp.float32),
            jax.ShapeDtypeStruct((G, H), jnp.float32),
        ],
        scratch_shapes=[pltpu.VMEM((N2, H), jnp.float32)],
    )(z3, n2s3, emb, x, pxW, pxb, iew1, ieb1, iew2, ieb2,
      inw1, inb1, inw2, inb2, c0w, s2s, s2g)


def _edge_agg(m, src2d, dst2d, zeros_n):
    mesh = plsc.VectorSubcoreMesh(core_axis_name="c", subcore_axis_name="s")

    @functools.partial(
        pl.kernel,
        mesh=mesh,
        out_type=jax.ShapeDtypeStruct((2, N, H), jnp.float32),
        scratch_types=[
            pltpu.VMEM((EROW,), jnp.int32),
            pltpu.VMEM((EROW,), jnp.int32),
            pltpu.VMEM((EROW,), jnp.int32),
            pltpu.VMEM((EROW,), jnp.int32),
            pltpu.VMEM((EROW, H), jnp.float32),
            pltpu.VMEM((EROW, H), jnp.float32),
            pltpu.VMEM_SHARED((N, H), jnp.float32),
            pltpu.SemaphoreType.DMA,
            pltpu.SemaphoreType.DMA,
        ],
    )
    def k(m_hbm, src_hbm, dst_hbm, z_hbm, out_hbm, src_a, dst_a, src_b,
          dst_b, rows_a, rows_b, acc, sem, sem_b):
        c = lax.axis_index("c")
        s = lax.axis_index("s")
        w = s * 2 + c
        rpt = 624
        tail = N - 16 * rpt

        pltpu.sync_copy(z_hbm.at[pl.ds(s * rpt, rpt)],
                        acc.at[pl.ds(s * rpt, rpt)])

        @pl.when(s == 15)
        def _():
            pltpu.sync_copy(z_hbm.at[pl.ds(16 * rpt, tail)],
                            acc.at[pl.ds(16 * rpt, tail)])

        plsc.subcore_barrier()

        def load_idx(r, sv, dv):
            pltpu.sync_copy(src_hbm.at[r], sv)
            pltpu.sync_copy(dst_hbm.at[r], dv)

        base = w * RPW
        last = base + RPW - 1
        load_idx(base, src_a, dst_a)
        pltpu.async_copy(m_hbm.at[src_a], rows_a, sem)

        def body(t, carry):
            ga = base + 2 * t
            load_idx(ga + 1, src_b, dst_b)
            pltpu.make_async_copy(m_hbm.at[src_a], rows_a, sem).wait()
            cpb = pltpu.async_copy(m_hbm.at[src_b], rows_b, sem_b)
            pltpu.sync_copy(rows_a, acc.at[dst_a], add=True)
            load_idx(jnp.minimum(ga + 2, last), src_a, dst_a)
            cpb.wait()
            pltpu.async_copy(m_hbm.at[src_a], rows_a, sem)
            pltpu.sync_copy(rows_b, acc.at[dst_b], add=True)
            return carry

        lax.fori_loop(0, RPW // 2, body, 0)
        pltpu.make_async_copy(m_hbm.at[src_a], rows_a, sem).wait()

        @pl.when(w < REM)
        def _():
            load_idx(NW * RPW + w, src_a, dst_a)
            pltpu.async_copy(m_hbm.at[src_a], rows_a, sem).wait()
            pltpu.sync_copy(rows_a, acc.at[dst_a], add=True)

        plsc.subcore_barrier()
        pltpu.sync_copy(acc.at[pl.ds(s * rpt, rpt)],
                        out_hbm.at[c, pl.ds(s * rpt, rpt)])

        @pl.when(s == 15)
        def _():
            pltpu.sync_copy(acc.at[pl.ds(16 * rpt, tail)],
                            out_hbm.at[c, pl.ds(16 * rpt, tail)])

    return k(m, src2d, dst2d, zeros_n)


def _conv_body(has_next, final, *refs):
    if final:
        (h_ref, p_ref, seg_ref, wih_ref, whh_ref, bih_ref, bhh_ref,
         ew1_ref, eb1_ref, ew2_ref, eb2_ref,
         nw1_ref, nb1_ref, nw2_ref, nb2_ref,
         xf_ref, s2s_ref, s2g_ref, ge0_ref, ge1_ref,
         pw1_ref, pb1_ref, pw2_ref, pb2_ref,
         pred_ref, acc_ref) = refs
    else:
        (h_ref, p_ref, seg_ref, wih_ref, whh_ref, bih_ref, bhh_ref,
         ew1_ref, eb1_ref, ew2_ref, eb2_ref,
         nw1_ref, nb1_ref, nw2_ref, nb2_ref,
         xf_ref, s2s_ref, s2g_ref, cwn_ref,
         hn_ref, mn_ref, ge_ref, acc_ref) = refs

    i = pl.program_id(0)

    @pl.when(i == 0)
    def _():
        acc_ref[...] = jnp.zeros_like(acc_ref)

    h = h_ref[...]
    agg = p_ref[0] + p_ref[1]
    gi = _mm(agg, wih_ref[...], ((1,), (1,))) + bih_ref[...]
    gh = _mm(h, whh_ref[...], ((1,), (1,))) + bhh_ref[...]
    r = _sig(gi[:, :H] + gh[:, :H])
    u = _sig(gi[:, H:2 * H] + gh[:, H:2 * H])
    nn_ = _tanh(gi[:, 2 * H:] + r * gh[:, 2 * H:])
    hn = jnp.maximum((1.0 - u) * nn_ + u * h, 0.0)

    if not final:
        hn_ref[...] = hn
    if has_next:
        mn_ref[...] = _mm(hn, cwn_ref[...], ((1,), (0,)))

    _seg_accum(acc_ref, seg_ref[0], hn, N2)

    @pl.when(i == NBLK - 1)
    def _():
        ne = _mlp2(acc_ref[...], ew1_ref[...], eb1_ref[...],
                   ew2_ref[...], eb2_ref[...])
        se = _mm1(_seg_onehot(s2s_ref[...], NS), ne, ((1,), (0,)))
        se = _mlp2(se, nw1_ref[...], nb1_ref[...],
                   nw2_ref[...], nb2_ref[...])
        ge = _mm1(_seg_onehot(s2g_ref[...], G), se * xf_ref[...],
                 ((1,), (0,)))
        if final:
            embed = ge0_ref[...] + ge1_ref[...] + ge
            hid = jnp.maximum(_mm(embed, pw1_ref[...], ((1,), (0,)))
                              + pb1_ref[...], 0.0)
            pred = _mm(hid, pw2_ref[...], ((1,), (0,))) + pb2_ref[...]
            mx = jnp.max(pred, axis=-1, keepdims=True)
            sh = pred - mx
            lse = jnp.log(jnp.sum(jnp.exp(sh), axis=-1, keepdims=True))
            pred_ref[...] = sh - lse
        else:
            ge_ref[...] = ge


def _conv_layer(h, parts, n2s3, wih, whh, bih, bhh,
                ew1, eb1, ew2, eb2, nw1, nb1, nw2, nb2,
                xf, s2s, s2g, cwn=None, finals=None):
    full = lambda shp: pl.BlockSpec(shp, lambda i: (0,) * len(shp))
    final = finals is not None
    has_next = cwn is not None
    in_specs = [
        pl.BlockSpec((NB, H), lambda i: (i, 0)),
        pl.BlockSpec((2, NB, H), lambda i: (0, i, 0)),
        pl.BlockSpec((1, 1, NB), lambda i: (i, 0, 0)),
        full((3 * H, H)), full((3 * H, H)), full((1, 3 * H)), full((1, 3 * H)),
        full((H, 2 * H)), full((1, 2 * H)), full((2 * H, H)), full((1, H)),
        full((H, 2 * H)), full((1, 2 * H)), full((2 * H, H)), full((1, H)),
        full((NS, H)), full((1, N2)), full((1, NS)),
    ]
    args = [h, parts, n2s3, wih, whh, bih, bhh,
            ew1, eb1, ew2, eb2, nw1, nb1, nw2, nb2, xf, s2s, s2g]
    if final:
        ge0, ge1, pw1, pb1, pw2, pb2 = finals
        in_specs += [full((G, H)), full((G, H)),
                     full((H, H)), full((1, H)), full((H, C)), full((1, C))]
        args += [ge0, ge1, pw1, pb1, pw2, pb2]
        out_specs = [full((G, C))]
        out_shape = [jax.ShapeDtypeStruct((G, C), jnp.float32)]
    else:
        in_specs += [full((H, H))]
        args += [cwn]
        out_specs = [pl.BlockSpec((NB, H), lambda i: (i, 0)),
                     pl.BlockSpec((NB, H), lambda i: (i, 0)),
                     full((G, H))]
        out_shape = [jax.ShapeDtypeStruct((N, H), jnp.float32),
                     jax.ShapeDtypeStruct((N, H), jnp.float32),
                     jax.ShapeDtypeStruct((G, H), jnp.float32)]
    return pl.pallas_call(
        functools.partial(_conv_body, has_next, final),
        grid=(NBLK,),
        in_specs=in_specs,
        out_specs=out_specs,
        out_shape=out_shape,
        scratch_shapes=[pltpu.VMEM((N2, H), jnp.float32)],
    )(*args)


def kernel(z, x, edge_index, batch, node_to_subgraph2, subgraph2_to_subgraph,
           subgraph_to_graph, emb, pxW, pxb, ie_w1, ie_b1, ie_w2, ie_b2,
           in_w1, in_b1, in_w2, in_b2,
           conv0_w, conv0_wih, conv0_whh, conv0_bih, conv0_bhh,
           e0_w1, e0_b1, e0_w2, e0_b2, n0_w1, n0_b1, n0_w2, n0_b2,
           conv1_w, conv1_wih, conv1_whh, conv1_bih, conv1_bhh,
           e1_w1, e1_b1, e1_w2, e1_b2, n1_w1, n1_b1, n1_w2, n1_b2,
           post_w1, post_b1, post_w2, post_b2):
    i32 = jnp.int32
    z3 = z.astype(i32).reshape(NBLK, 1, NB)
    n2s3 = node_to_subgraph2.astype(i32).reshape(NBLK, 1, NB)
    s2s = subgraph2_to_subgraph.astype(i32).reshape(1, N2)
    s2g = subgraph_to_graph.astype(i32).reshape(1, NS)
    src2d = edge_index[0].astype(i32).reshape(ER, EROW)
    dst2d = edge_index[1].astype(i32).reshape(ER, EROW)
    zeros_n = jnp.zeros((N, H), jnp.float32)
    row = lambda b: b.reshape(1, -1)

    zf, m0, xf, ge0 = _prelude(
        z3, n2s3, emb, x, pxW, row(pxb),
        ie_w1, row(ie_b1), ie_w2, row(ie_b2),
        in_w1, row(in_b1), in_w2, row(in_b2), conv0_w, s2s, s2g)

    parts0 = _edge_agg(m0, src2d, dst2d, zeros_n)
    h1, m1, ge1 = _conv_layer(
        zf, parts0, n2s3, conv0_wih, conv0_whh, row(conv0_bih),
        row(conv0_bhh), e0_w1, row(e0_b1), e0_w2, row(e0_b2),
        n0_w1, row(n0_b1), n0_w2, row(n0_b2), xf, s2s, s2g, cwn=conv1_w)

    parts1 = _edge_agg(m1, src2d, dst2d, zeros_n)
    (pred,) = _conv_layer(
        h1, parts1, n2s3, conv1_wih, conv1_whh, row(conv1_bih),
        row(conv1_bhh), e1_w1, row(e1_b1), e1_w2, row(e1_b2),
        n1_w1, row(n1_b1), n1_w2, row(n1_b2), xf, s2s, s2g,
        finals=(ge0, ge1, post_w1, row(post_b1), post_w2, row(post_b2)))
    return pred
